# Initial kernel scaffold; baseline (speedup 1.0000x reference)
import jax
import jax.numpy as jnp
from jax.experimental import pallas as pl


def kernel(z, pos, batch, emb, mlp_w1, mlp_b1, mlp_w2, mlp_b2, cf_w1, cf_w2, cf_b2, lin_w, lin_b, lin1_w, lin1_b, lin2_w, lin2_b):
    raise NotImplementedError("write your pallas kernel here")


# speedup vs baseline: 11.4346x; 11.4346x over previous
import functools
import math

import jax
import jax.numpy as jnp
import numpy as np
from jax.experimental import pallas as pl
from jax.experimental.pallas import tpu as pltpu

_HIDDEN = 128
_FILTERS = 128
_NUM_INT = 6
_NUM_G = 50
_CUTOFF = 10.0
_N = 8192
_NG = 1024

_BLK = 32
_W = 128
_DB = 512

_LOG2 = math.log(2.0)
_OFFSET = np.linspace(0.0, _CUTOFF, _NUM_G).astype(np.float32)
_COEFF = float(-0.5 / (_OFFSET[1] - _OFFSET[0]) ** 2)


def _ssp(x):
    return jax.nn.softplus(x) - _LOG2


def _embed_kernel(z_ref, emb_ref, w1_ref, h_ref, xl_ref):
    zb = z_ref[...]
    ids = jax.lax.broadcasted_iota(jnp.int32, (1, 100), 1)
    onehot = (zb == ids).astype(jnp.float32)
    h = jnp.dot(onehot, emb_ref[...], preferred_element_type=jnp.float32)
    h_ref[...] = h
    xl_ref[...] = jnp.dot(h, w1_ref[...], preferred_element_type=jnp.float32)


def _msg_kernel(lo_ref, hi_ref, pos_ref, batch_ref, xl_ref,
                w1_ref, b1_ref, w2_ref, b2_ref, out_ref):
    b = pl.program_id(0)
    k0 = lo_ref[b]
    k1 = hi_ref[b]
    base = b * _BLK
    offset = jax.lax.broadcasted_iota(jnp.int32, (1, _NUM_G), 1).astype(
        jnp.float32) * (_CUTOFF / (_NUM_G - 1))
    w1 = w1_ref[...]
    b1 = b1_ref[...]
    w2 = w2_ref[...]
    b2 = b2_ref[...]

    def row_body(r, carry):
        i = base + r
        p_r = pos_ref[pl.ds(i, 1), :]
        bat_r = batch_ref[pl.ds(i, 1), :]

        def win_body(k, acc):
            wb = k * _W
            pos_w = pos_ref[pl.ds(wb, _W), :]
            bat_w = batch_ref[pl.ds(wb, _W), :]
            xl_w = xl_ref[pl.ds(wb, _W), :]
            diff = pos_w - p_r
            d2 = jnp.sum(diff * diff, axis=1, keepdims=True)
            jidx = wb + jax.lax.broadcasted_iota(jnp.int32, (_W, 1), 0)
            mask = (d2 < _CUTOFF * _CUTOFF) & (bat_w == bat_r) & (jidx != i)
            ew = jnp.sqrt(d2)
            ea = jnp.exp(_COEFF * (ew - offset) ** 2)
            t = _ssp(jnp.dot(ea, w1, preferred_element_type=jnp.float32) + b1)
            wt = jnp.dot(t, w2, preferred_element_type=jnp.float32) + b2
            c = 0.5 * (jnp.cos(ew * (math.pi / _CUTOFF)) + 1.0)
            msg = jnp.where(mask, xl_w * (wt * c), 0.0)
            return acc + jnp.sum(msg, axis=0, keepdims=True)

        acc = jax.lax.fori_loop(k0, k1 + 1, win_body,
                                jnp.zeros((1, _HIDDEN), jnp.float32))
        out_ref[pl.ds(r, 1), :] = acc
        return carry

    jax.lax.fori_loop(0, _BLK, row_body, 0)


def _post_kernel(agg_ref, h_ref, w2c_ref, b2c_ref, lw_ref, lb_ref, w1n_ref,
                 hout_ref, xlout_ref):
    xc = jnp.dot(agg_ref[...], w2c_ref[...],
                 preferred_element_type=jnp.float32) + b2c_ref[...]
    xo = jnp.dot(_ssp(xc), lw_ref[...],
                 preferred_element_type=jnp.float32) + lb_ref[...]
    h = h_ref[...] + xo
    hout_ref[...] = h
    xlout_ref[...] = jnp.dot(h, w1n_ref[...], preferred_element_type=jnp.float32)


def _head_kernel(h_ref, l1w_ref, l1b_ref, l2w_ref, l2b_ref, batchT_ref, out_ref):
    j = pl.program_id(0)
    t = _ssp(jnp.dot(h_ref[...], l1w_ref[...],
                     preferred_element_type=jnp.float32) + l1b_ref[...])
    y = jnp.dot(t, l2w_ref[...], preferred_element_type=jnp.float32) + l2b_ref[...]
    g = jax.lax.broadcasted_iota(jnp.int32, (_NG, 1), 0)
    onehot_t = (batchT_ref[...] == g).astype(jnp.float32)

    @pl.when(j == 0)
    def _():
        out_ref[...] = jnp.zeros_like(out_ref)

    out_ref[...] += jnp.dot(onehot_t, y, preferred_element_type=jnp.float32)


def _full(shape):
    return pl.BlockSpec(shape, lambda b: tuple(0 for _ in shape))


def kernel(z, pos, batch, emb, mlp_w1, mlp_b1, mlp_w2, mlp_b2, cf_w1, cf_w2,
           cf_b2, lin_w, lin_b, lin1_w, lin1_b, lin2_w, lin2_b):
    batch = batch.astype(jnp.int32)
    z2 = z.astype(jnp.int32).reshape(_N, 1)
    batch2 = batch.reshape(_N, 1)
    batch_t = batch.reshape(1, _N)

    gids = jnp.arange(_NG, dtype=jnp.int32)
    seg_lo = jnp.searchsorted(batch, gids, side='left').astype(jnp.int32)
    seg_hi = jnp.searchsorted(batch, gids, side='right').astype(jnp.int32)
    first = batch[:: _BLK]
    last = batch[_BLK - 1:: _BLK]
    row_lo = seg_lo[first]
    row_hi = seg_hi[last]
    k_lo = (row_lo // _W).astype(jnp.int32)
    k_hi = ((row_hi - 1) // _W).astype(jnp.int32)

    nb = _N // _BLK
    nd = _N // _DB

    h, xl = pl.pallas_call(
        _embed_kernel,
        grid=(nd,),
        in_specs=[
            pl.BlockSpec((_DB, 1), lambda j: (j, 0)),
            _full((100, _HIDDEN)),
            _full((_HIDDEN, _FILTERS)),
        ],
        out_specs=[
            pl.BlockSpec((_DB, _HIDDEN), lambda j: (j, 0)),
            pl.BlockSpec((_DB, _FILTERS), lambda j: (j, 0)),
        ],
        out_shape=[
            jax.ShapeDtypeStruct((_N, _HIDDEN), jnp.float32),
            jax.ShapeDtypeStruct((_N, _FILTERS), jnp.float32),
        ],
    )(z2, emb, cf_w1[0])

    msg_call = pl.pallas_call(
        _msg_kernel,
        grid=(nb,),
        in_specs=[
            pl.BlockSpec(memory_space=pltpu.SMEM),
            pl.BlockSpec(memory_space=pltpu.SMEM),
            _full((_N, 3)),
            _full((_N, 1)),
            _full((_N, _FILTERS)),
            _full((_NUM_G, _FILTERS)),
            _full((1, _FILTERS)),
            _full((_FILTERS, _FILTERS)),
            _full((1, _FILTERS)),
        ],
        out_specs=pl.BlockSpec((_BLK, _FILTERS), lambda b: (b, 0)),
        out_shape=jax.ShapeDtypeStruct((_N, _FILTERS), jnp.float32),
    )

    post_call = pl.pallas_call(
        _post_kernel,
        grid=(nd,),
        in_specs=[
            pl.BlockSpec((_DB, _FILTERS), lambda j: (j, 0)),
            pl.BlockSpec((_DB, _HIDDEN), lambda j: (j, 0)),
            _full((_FILTERS, _HIDDEN)),
            _full((1, _HIDDEN)),
            _full((_HIDDEN, _HIDDEN)),
            _full((1, _HIDDEN)),
            _full((_HIDDEN, _FILTERS)),
        ],
        out_specs=[
            pl.BlockSpec((_DB, _HIDDEN), lambda j: (j, 0)),
            pl.BlockSpec((_DB, _FILTERS), lambda j: (j, 0)),
        ],
        out_shape=[
            jax.ShapeDtypeStruct((_N, _HIDDEN), jnp.float32),
            jax.ShapeDtypeStruct((_N, _FILTERS), jnp.float32),
        ],
    )

    for i in range(_NUM_INT):
        agg = msg_call(k_lo, k_hi, pos, batch2, xl,
                       mlp_w1[i], mlp_b1[i].reshape(1, _FILTERS),
                       mlp_w2[i], mlp_b2[i].reshape(1, _FILTERS))
        w1n = cf_w1[(i + 1) % _NUM_INT]
        h, xl = post_call(agg, h, cf_w2[i], cf_b2[i].reshape(1, _HIDDEN),
                          lin_w[i], lin_b[i].reshape(1, _HIDDEN), w1n)

    out = pl.pallas_call(
        _head_kernel,
        grid=(nd,),
        in_specs=[
            pl.BlockSpec((_DB, _HIDDEN), lambda j: (j, 0)),
            _full((_HIDDEN, _HIDDEN // 2)),
            _full((1, _HIDDEN // 2)),
            _full((_HIDDEN // 2, 1)),
            _full((1, 1)),
            pl.BlockSpec((1, _DB), lambda j: (0, j)),
        ],
        out_specs=pl.BlockSpec((_NG, 1), lambda j: (0, 0)),
        out_shape=jax.ShapeDtypeStruct((_NG, 1), jnp.float32),
    )(h, lin1_w, lin1_b.reshape(1, _HIDDEN // 2),
      lin2_w, lin2_b.reshape(1, 1), batch_t)

    return out


# speedup vs baseline: 43.4267x; 3.7978x over previous
import functools
import math

import jax
import jax.numpy as jnp
import numpy as np
from jax.experimental import pallas as pl
from jax.experimental.pallas import tpu as pltpu

_HIDDEN = 128
_FILTERS = 128
_NUM_INT = 6
_NUM_G = 50
_CUTOFF = 10.0
_N = 8192
_NG = 1024

_BLK = 256
_WP = 32
_U = 4
_DB = 512

_LOG2 = math.log(2.0)
_OFFSET = np.linspace(0.0, _CUTOFF, _NUM_G).astype(np.float32)
_COEFF = float(-0.5 / (_OFFSET[1] - _OFFSET[0]) ** 2)


def _ssp(x):
    return jax.nn.softplus(x) - _LOG2


def _embed_kernel(z_ref, emb_ref, w1_ref, h_ref, xl_ref):
    zb = z_ref[...]
    ids = jax.lax.broadcasted_iota(jnp.int32, (1, 100), 1)
    onehot = (zb == ids).astype(jnp.float32)
    h = jnp.dot(onehot, emb_ref[...], preferred_element_type=jnp.float32)
    h_ref[...] = h
    xl_ref[...] = jnp.dot(h, w1_ref[...], preferred_element_type=jnp.float32)


def _msg_kernel(lo_ref, nw_ref, pos_ref, batch_ref, xl_ref,
                w1_ref, b1_ref, w2_ref, b2_ref, out_ref):
    b = pl.program_id(0)
    base = b * _BLK
    offset = jax.lax.broadcasted_iota(jnp.int32, (1, _NUM_G), 1).astype(
        jnp.float32) * (_CUTOFF / (_NUM_G - 1))
    w1 = w1_ref[...]
    b1 = b1_ref[...]
    w2 = w2_ref[...]
    b2 = b2_ref[...]

    def pair_block(i, xs, uc):
        pos_w = pos_ref[pl.ds(xs, _WP), :]
        p_r = pos_ref[pl.ds(i, 1), :]
        bat_w = batch_ref[pl.ds(xs, _WP), :]
        bat_r = batch_ref[pl.ds(i, 1), :]
        diff = pos_w - p_r
        d2 = jnp.sum(diff * diff, axis=1, keepdims=True)
        jidx = xs + jax.lax.broadcasted_iota(jnp.int32, (_WP, 1), 0)
        mask = ((d2 < _CUTOFF * _CUTOFF) & (bat_w == bat_r)
                & (jidx != i) & (jidx >= uc))
        return d2, mask, xl_ref[pl.ds(xs, _WP), :]

    def filt(d2, mask, xl_w):
        ew = jnp.sqrt(d2)
        ea = jnp.exp(_COEFF * (ew - offset) ** 2)
        t = _ssp(jnp.dot(ea, w1, preferred_element_type=jnp.float32) + b1)
        wt = jnp.dot(t, w2, preferred_element_type=jnp.float32) + b2
        c = 0.5 * (jnp.cos(ew * (math.pi / _CUTOFF)) + 1.0)
        return jnp.where(mask, xl_w * (wt * c), 0.0)

    def group_body(g, carry):
        r0 = g * _U
        d2s, masks, xls, nws, los = [], [], [], [], []
        for u in range(_U):
            i = base + r0 + u
            lo = lo_ref[i]
            los.append(lo)
            nws.append(nw_ref[i])
            xs = jnp.minimum(lo, _N - _WP)
            d2, m, xw = pair_block(i, xs, 0)
            d2s.append(d2)
            masks.append(m)
            xls.append(xw)
        msg = filt(jnp.concatenate(d2s, axis=0),
                   jnp.concatenate(masks, axis=0),
                   jnp.concatenate(xls, axis=0))
        for u in range(_U):
            part = jnp.sum(msg[u * _WP:(u + 1) * _WP, :], axis=0,
                           keepdims=True)
            out_ref[pl.ds(r0 + u, 1), :] = part

        nw_max = functools.reduce(jnp.maximum, nws)

        @pl.when(nw_max > 1)
        def _rest():
            for u in range(_U):
                i = base + r0 + u
                lo = los[u]

                def win_body(k, acc, i=i, lo=lo):
                    uc = lo + k * _WP
                    xs = jnp.minimum(uc, _N - _WP)
                    d2, m, xw = pair_block(i, xs, uc)
                    msg1 = filt(d2, m, xw)
                    return acc + jnp.sum(msg1, axis=0, keepdims=True)

                extra = jax.lax.fori_loop(
                    1, nws[u], win_body, jnp.zeros((1, _HIDDEN), jnp.float32))
                out_ref[pl.ds(r0 + u, 1), :] += extra

        return carry

    jax.lax.fori_loop(0, _BLK // _U, group_body, 0)


def _post_kernel(agg_ref, h_ref, w2c_ref, b2c_ref, lw_ref, lb_ref, w1n_ref,
                 hout_ref, xlout_ref):
    xc = jnp.dot(agg_ref[...], w2c_ref[...],
                 preferred_element_type=jnp.float32) + b2c_ref[...]
    xo = jnp.dot(_ssp(xc), lw_ref[...],
                 preferred_element_type=jnp.float32) + lb_ref[...]
    h = h_ref[...] + xo
    hout_ref[...] = h
    xlout_ref[...] = jnp.dot(h, w1n_ref[...], preferred_element_type=jnp.float32)


def _head_kernel(h_ref, l1w_ref, l1b_ref, l2w_ref, l2b_ref, batchT_ref, out_ref):
    j = pl.program_id(0)
    t = _ssp(jnp.dot(h_ref[...], l1w_ref[...],
                     preferred_element_type=jnp.float32) + l1b_ref[...])
    y = jnp.dot(t, l2w_ref[...], preferred_element_type=jnp.float32) + l2b_ref[...]
    g = jax.lax.broadcasted_iota(jnp.int32, (_NG, 1), 0)
    onehot_t = (batchT_ref[...] == g).astype(jnp.float32)

    @pl.when(j == 0)
    def _():
        out_ref[...] = jnp.zeros_like(out_ref)

    out_ref[...] += jnp.dot(onehot_t, y, preferred_element_type=jnp.float32)


def _full(shape):
    return pl.BlockSpec(shape, lambda b: tuple(0 for _ in shape))


def kernel(z, pos, batch, emb, mlp_w1, mlp_b1, mlp_w2, mlp_b2, cf_w1, cf_w2,
           cf_b2, lin_w, lin_b, lin1_w, lin1_b, lin2_w, lin2_b):
    batch = batch.astype(jnp.int32)
    z2 = z.astype(jnp.int32).reshape(_N, 1)
    batch2 = batch.reshape(_N, 1)
    batch_t = batch.reshape(1, _N)

    gids = jnp.arange(_NG, dtype=jnp.int32)
    seg_lo = jnp.searchsorted(batch, gids, side='left').astype(jnp.int32)
    seg_hi = jnp.searchsorted(batch, gids, side='right').astype(jnp.int32)
    node_lo = seg_lo[batch]
    node_len = seg_hi[batch] - node_lo
    node_nw = (node_len + _WP - 1) // _WP

    nb = _N // _BLK
    nd = _N // _DB

    h, xl = pl.pallas_call(
        _embed_kernel,
        grid=(nd,),
        in_specs=[
            pl.BlockSpec((_DB, 1), lambda j: (j, 0)),
            _full((100, _HIDDEN)),
            _full((_HIDDEN, _FILTERS)),
        ],
        out_specs=[
            pl.BlockSpec((_DB, _HIDDEN), lambda j: (j, 0)),
            pl.BlockSpec((_DB, _FILTERS), lambda j: (j, 0)),
        ],
        out_shape=[
            jax.ShapeDtypeStruct((_N, _HIDDEN), jnp.float32),
            jax.ShapeDtypeStruct((_N, _FILTERS), jnp.float32),
        ],
    )(z2, emb, cf_w1[0])

    msg_call = pl.pallas_call(
        _msg_kernel,
        grid=(nb,),
        in_specs=[
            pl.BlockSpec(memory_space=pltpu.SMEM),
            pl.BlockSpec(memory_space=pltpu.SMEM),
            _full((_N, 3)),
            _full((_N, 1)),
            _full((_N, _FILTERS)),
            _full((_NUM_G, _FILTERS)),
            _full((1, _FILTERS)),
            _full((_FILTERS, _FILTERS)),
            _full((1, _FILTERS)),
        ],
        out_specs=pl.BlockSpec((_BLK, _FILTERS), lambda b: (b, 0)),
        out_shape=jax.ShapeDtypeStruct((_N, _FILTERS), jnp.float32),
    )

    post_call = pl.pallas_call(
        _post_kernel,
        grid=(nd,),
        in_specs=[
            pl.BlockSpec((_DB, _FILTERS), lambda j: (j, 0)),
            pl.BlockSpec((_DB, _HIDDEN), lambda j: (j, 0)),
            _full((_FILTERS, _HIDDEN)),
            _full((1, _HIDDEN)),
            _full((_HIDDEN, _HIDDEN)),
            _full((1, _HIDDEN)),
            _full((_HIDDEN, _FILTERS)),
        ],
        out_specs=[
            pl.BlockSpec((_DB, _HIDDEN), lambda j: (j, 0)),
            pl.BlockSpec((_DB, _FILTERS), lambda j: (j, 0)),
        ],
        out_shape=[
            jax.ShapeDtypeStruct((_N, _HIDDEN), jnp.float32),
            jax.ShapeDtypeStruct((_N, _FILTERS), jnp.float32),
        ],
    )

    for i in range(_NUM_INT):
        agg = msg_call(node_lo, node_nw, pos, batch2, xl,
                       mlp_w1[i], mlp_b1[i].reshape(1, _FILTERS),
                       mlp_w2[i], mlp_b2[i].reshape(1, _FILTERS))
        w1n = cf_w1[(i + 1) % _NUM_INT]
        h, xl = post_call(agg, h, cf_w2[i], cf_b2[i].reshape(1, _HIDDEN),
                          lin_w[i], lin_b[i].reshape(1, _HIDDEN), w1n)

    out = pl.pallas_call(
        _head_kernel,
        grid=(nd,),
        in_specs=[
            pl.BlockSpec((_DB, _HIDDEN), lambda j: (j, 0)),
            _full((_HIDDEN, _HIDDEN // 2)),
            _full((1, _HIDDEN // 2)),
            _full((_HIDDEN // 2, 1)),
            _full((1, 1)),
            pl.BlockSpec((1, _DB), lambda j: (0, j)),
        ],
        out_specs=pl.BlockSpec((_NG, 1), lambda j: (0, 0)),
        out_shape=jax.ShapeDtypeStruct((_NG, 1), jnp.float32),
    )(h, lin1_w, lin1_b.reshape(1, _HIDDEN // 2),
      lin2_w, lin2_b.reshape(1, 1), batch_t)

    return out


# speedup vs baseline: 54.4976x; 1.2549x over previous
import functools
import math

import jax
import jax.numpy as jnp
import numpy as np
from jax.experimental import pallas as pl
from jax.experimental.pallas import tpu as pltpu

_HIDDEN = 128
_FILTERS = 128
_NUM_INT = 6
_NUM_G = 50
_CUTOFF = 10.0
_N = 8192
_NG = 1024

_BLK = 256
_WP = 32
_U = 8
_G2 = 2
_DB = 512

_LOG2 = math.log(2.0)
_OFFSET = np.linspace(0.0, _CUTOFF, _NUM_G).astype(np.float32)
_COEFF = float(-0.5 / (_OFFSET[1] - _OFFSET[0]) ** 2)


def _ssp(x):
    return jax.nn.softplus(x) - _LOG2


def _embed_kernel(z_ref, emb_ref, w1_ref, h_ref, xl_ref):
    zb = z_ref[...]
    ids = jax.lax.broadcasted_iota(jnp.int32, (1, 100), 1)
    onehot = (zb == ids).astype(jnp.float32)
    h = jnp.dot(onehot, emb_ref[...], preferred_element_type=jnp.float32)
    h_ref[...] = h
    xl_ref[...] = jnp.dot(h, w1_ref[...], preferred_element_type=jnp.float32)


def _msg_kernel(lo_ref, nw_ref, pos_ref, batch_ref, xl_ref,
                w1_ref, b1_ref, w2_ref, b2_ref, out_ref):
    b = pl.program_id(0)
    base = b * _BLK
    offset = jax.lax.broadcasted_iota(jnp.int32, (1, _NUM_G), 1).astype(
        jnp.float32) * (_CUTOFF / (_NUM_G - 1))
    w1 = w1_ref[...]
    b1 = b1_ref[...]
    w2 = w2_ref[...]
    b2 = b2_ref[...]

    def pair_block(i, xs, uc):
        pos_w = pos_ref[pl.ds(xs, _WP), :]
        p_r = pos_ref[pl.ds(i, 1), :]
        bat_w = batch_ref[pl.ds(xs, _WP), :]
        bat_r = batch_ref[pl.ds(i, 1), :]
        diff = pos_w - p_r
        d2 = jnp.sum(diff * diff, axis=1, keepdims=True)
        jidx = xs + jax.lax.broadcasted_iota(jnp.int32, (_WP, 1), 0)
        mask = ((d2 < _CUTOFF * _CUTOFF) & (bat_w == bat_r)
                & (jidx != i) & (jidx >= uc))
        return d2, mask, xl_ref[pl.ds(xs, _WP), :]

    sel = (jax.lax.broadcasted_iota(jnp.int32, (_U, _U * _WP), 1) // _WP ==
           jax.lax.broadcasted_iota(jnp.int32, (_U, _U * _WP), 0)
           ).astype(jnp.float32)

    def filt(d2, mask, xl_w):
        ew = jnp.sqrt(d2)
        ea = jnp.exp(_COEFF * (ew - offset) ** 2)
        t = _ssp(jnp.dot(ea, w1, preferred_element_type=jnp.float32) + b1)
        cm = jnp.where(mask, 0.5 * (jnp.cos(ew * (math.pi / _CUTOFF)) + 1.0),
                       0.0)
        wtm = jnp.dot(t * cm, w2, preferred_element_type=jnp.float32) + b2 * cm
        return xl_w * wtm

    def group(g):
        r0 = g * _U
        d2s, masks, xls, nws, los = [], [], [], [], []
        for u in range(_U):
            i = base + r0 + u
            lo = lo_ref[i]
            los.append(lo)
            nws.append(nw_ref[i])
            xs = jnp.minimum(lo, _N - _WP)
            d2, m, xw = pair_block(i, xs, 0)
            d2s.append(d2)
            masks.append(m)
            xls.append(xw)
        msg = filt(jnp.concatenate(d2s, axis=0),
                   jnp.concatenate(masks, axis=0),
                   jnp.concatenate(xls, axis=0))
        out_ref[pl.ds(r0, _U), :] = jnp.dot(
            sel, msg, preferred_element_type=jnp.float32)

        nw_max = functools.reduce(jnp.maximum, nws)

        @pl.when(nw_max > 1)
        def _rest():
            for u in range(_U):
                i = base + r0 + u
                lo = los[u]

                def win_body(k, acc, i=i, lo=lo):
                    uc = lo + k * _WP
                    xs = jnp.minimum(uc, _N - _WP)
                    d2, m, xw = pair_block(i, xs, uc)
                    msg1 = filt(d2, m, xw)
                    return acc + jnp.sum(msg1, axis=0, keepdims=True)

                extra = jax.lax.fori_loop(
                    1, nws[u], win_body, jnp.zeros((1, _HIDDEN), jnp.float32))
                out_ref[pl.ds(r0 + u, 1), :] += extra

    def body(gg, carry):
        for s in range(_G2):
            group(gg * _G2 + s)
        return carry

    jax.lax.fori_loop(0, _BLK // (_U * _G2), body, 0)


def _post_kernel(agg_ref, h_ref, w2c_ref, b2c_ref, lw_ref, lb_ref, w1n_ref,
                 hout_ref, xlout_ref):
    xc = jnp.dot(agg_ref[...], w2c_ref[...],
                 preferred_element_type=jnp.float32) + b2c_ref[...]
    xo = jnp.dot(_ssp(xc), lw_ref[...],
                 preferred_element_type=jnp.float32) + lb_ref[...]
    h = h_ref[...] + xo
    hout_ref[...] = h
    xlout_ref[...] = jnp.dot(h, w1n_ref[...], preferred_element_type=jnp.float32)


def _head_kernel(h_ref, l1w_ref, l1b_ref, l2w_ref, l2b_ref, batchT_ref, out_ref):
    j = pl.program_id(0)
    t = _ssp(jnp.dot(h_ref[...], l1w_ref[...],
                     preferred_element_type=jnp.float32) + l1b_ref[...])
    y = jnp.dot(t, l2w_ref[...], preferred_element_type=jnp.float32) + l2b_ref[...]
    g = jax.lax.broadcasted_iota(jnp.int32, (_NG, 1), 0)
    onehot_t = (batchT_ref[...] == g).astype(jnp.float32)

    @pl.when(j == 0)
    def _():
        out_ref[...] = jnp.zeros_like(out_ref)

    out_ref[...] += jnp.dot(onehot_t, y, preferred_element_type=jnp.float32)


def _full(shape):
    return pl.BlockSpec(shape, lambda b: tuple(0 for _ in shape))


def kernel(z, pos, batch, emb, mlp_w1, mlp_b1, mlp_w2, mlp_b2, cf_w1, cf_w2,
           cf_b2, lin_w, lin_b, lin1_w, lin1_b, lin2_w, lin2_b):
    batch = batch.astype(jnp.int32)
    z2 = z.astype(jnp.int32).reshape(_N, 1)
    batch2 = batch.reshape(_N, 1)
    batch_t = batch.reshape(1, _N)

    gids = jnp.arange(_NG, dtype=jnp.int32)
    seg_lo = jnp.searchsorted(batch, gids, side='left').astype(jnp.int32)
    seg_hi = jnp.searchsorted(batch, gids, side='right').astype(jnp.int32)
    node_lo = seg_lo[batch]
    node_len = seg_hi[batch] - node_lo
    node_nw = (node_len + _WP - 1) // _WP

    nb = _N // _BLK
    nd = _N // _DB

    h, xl = pl.pallas_call(
        _embed_kernel,
        grid=(nd,),
        in_specs=[
            pl.BlockSpec((_DB, 1), lambda j: (j, 0)),
            _full((100, _HIDDEN)),
            _full((_HIDDEN, _FILTERS)),
        ],
        out_specs=[
            pl.BlockSpec((_DB, _HIDDEN), lambda j: (j, 0)),
            pl.BlockSpec((_DB, _FILTERS), lambda j: (j, 0)),
        ],
        out_shape=[
            jax.ShapeDtypeStruct((_N, _HIDDEN), jnp.float32),
            jax.ShapeDtypeStruct((_N, _FILTERS), jnp.float32),
        ],
    )(z2, emb, cf_w1[0])

    msg_call = pl.pallas_call(
        _msg_kernel,
        grid=(nb,),
        in_specs=[
            pl.BlockSpec(memory_space=pltpu.SMEM),
            pl.BlockSpec(memory_space=pltpu.SMEM),
            _full((_N, 3)),
            _full((_N, 1)),
            _full((_N, _FILTERS)),
            _full((_NUM_G, _FILTERS)),
            _full((1, _FILTERS)),
            _full((_FILTERS, _FILTERS)),
            _full((1, _FILTERS)),
        ],
        out_specs=pl.BlockSpec((_BLK, _FILTERS), lambda b: (b, 0)),
        out_shape=jax.ShapeDtypeStruct((_N, _FILTERS), jnp.float32),
    )

    post_call = pl.pallas_call(
        _post_kernel,
        grid=(nd,),
        in_specs=[
            pl.BlockSpec((_DB, _FILTERS), lambda j: (j, 0)),
            pl.BlockSpec((_DB, _HIDDEN), lambda j: (j, 0)),
            _full((_FILTERS, _HIDDEN)),
            _full((1, _HIDDEN)),
            _full((_HIDDEN, _HIDDEN)),
            _full((1, _HIDDEN)),
            _full((_HIDDEN, _FILTERS)),
        ],
        out_specs=[
            pl.BlockSpec((_DB, _HIDDEN), lambda j: (j, 0)),
            pl.BlockSpec((_DB, _FILTERS), lambda j: (j, 0)),
        ],
        out_shape=[
            jax.ShapeDtypeStruct((_N, _HIDDEN), jnp.float32),
            jax.ShapeDtypeStruct((_N, _FILTERS), jnp.float32),
        ],
    )

    for i in range(_NUM_INT):
        agg = msg_call(node_lo, node_nw, pos, batch2, xl,
                       mlp_w1[i], mlp_b1[i].reshape(1, _FILTERS),
                       mlp_w2[i], mlp_b2[i].reshape(1, _FILTERS))
        w1n = cf_w1[(i + 1) % _NUM_INT]
        h, xl = post_call(agg, h, cf_w2[i], cf_b2[i].reshape(1, _HIDDEN),
                          lin_w[i], lin_b[i].reshape(1, _HIDDEN), w1n)

    out = pl.pallas_call(
        _head_kernel,
        grid=(nd,),
        in_specs=[
            pl.BlockSpec((_DB, _HIDDEN), lambda j: (j, 0)),
            _full((_HIDDEN, _HIDDEN // 2)),
            _full((1, _HIDDEN // 2)),
            _full((_HIDDEN // 2, 1)),
            _full((1, 1)),
            pl.BlockSpec((1, _DB), lambda j: (0, j)),
        ],
        out_specs=pl.BlockSpec((_NG, 1), lambda j: (0, 0)),
        out_shape=jax.ShapeDtypeStruct((_NG, 1), jnp.float32),
    )(h, lin1_w, lin1_b.reshape(1, _HIDDEN // 2),
      lin2_w, lin2_b.reshape(1, 1), batch_t)

    return out


# speedup vs baseline: 54.5781x; 1.0015x over previous
import functools
import math

import jax
import jax.numpy as jnp
import numpy as np
from jax.experimental import pallas as pl
from jax.experimental.pallas import tpu as pltpu

_HIDDEN = 128
_FILTERS = 128
_NUM_INT = 6
_NUM_G = 50
_CUTOFF = 10.0
_N = 8192
_NG = 1024

_BLK = 256
_WP = 32
_U = 8
_G2 = 2
_DB = 512

_LOG2 = math.log(2.0)
_OFFSET = np.linspace(0.0, _CUTOFF, _NUM_G).astype(np.float32)
_COEFF = float(-0.5 / (_OFFSET[1] - _OFFSET[0]) ** 2)


def _ssp(x):
    return jax.nn.softplus(x) - _LOG2


def _embed_kernel(z_ref, emb_ref, w1_ref, h_ref, xl_ref):
    zb = z_ref[...]
    ids = jax.lax.broadcasted_iota(jnp.int32, (1, 100), 1)
    onehot = (zb == ids).astype(jnp.float32)
    h = jnp.dot(onehot, emb_ref[...], preferred_element_type=jnp.float32)
    h_ref[...] = h
    xl_ref[...] = jnp.dot(h, w1_ref[...], preferred_element_type=jnp.float32)


def _msg_kernel(lo_ref, nw_ref, pos_ref, batch_ref, xl_ref,
                w1_ref, b1_ref, w2_ref, b2_ref, out_ref):
    b = pl.program_id(0)
    base = b * _BLK
    offset = jax.lax.broadcasted_iota(jnp.int32, (1, _NUM_G), 1).astype(
        jnp.float32) * (_CUTOFF / (_NUM_G - 1))
    w1 = w1_ref[...]
    b1 = b1_ref[...]
    w2 = w2_ref[...]
    b2 = b2_ref[...]

    def pair_block(i, xs, uc):
        pos_w = pos_ref[pl.ds(xs, _WP), :]
        p_r = pos_ref[pl.ds(i, 1), :]
        bat_w = batch_ref[pl.ds(xs, _WP), :]
        bat_r = batch_ref[pl.ds(i, 1), :]
        diff = pos_w - p_r
        d2 = jnp.sum(diff * diff, axis=1, keepdims=True)
        mask = (d2 < _CUTOFF * _CUTOFF) & (bat_w == bat_r)
        if uc is not None:
            jidx = xs + jax.lax.broadcasted_iota(jnp.int32, (_WP, 1), 0)
            mask = mask & (jidx >= uc)
        return d2, mask, xl_ref[pl.ds(xs, _WP), :]

    sel = (jax.lax.broadcasted_iota(jnp.int32, (_U, _U * _WP), 1) // _WP ==
           jax.lax.broadcasted_iota(jnp.int32, (_U, _U * _WP), 0)
           ).astype(jnp.float32)

    ea0 = jnp.exp(_COEFF * offset ** 2)
    t0 = _ssp(jnp.dot(ea0, w1, preferred_element_type=jnp.float32) + b1)
    wt0 = jnp.dot(t0, w2, preferred_element_type=jnp.float32) + b2

    def filt(d2, mask, xl_w):
        ew = jnp.sqrt(d2)
        ea = jnp.exp(_COEFF * (ew - offset) ** 2)
        t = _ssp(jnp.dot(ea, w1, preferred_element_type=jnp.float32) + b1)
        cm = jnp.where(mask, 0.5 * (jnp.cos(ew * (math.pi / _CUTOFF)) + 1.0),
                       0.0)
        wtm = jnp.dot(t * cm, w2, preferred_element_type=jnp.float32) + b2 * cm
        return xl_w * wtm

    def group(g):
        r0 = g * _U
        d2s, masks, xls, nws, los = [], [], [], [], []
        for u in range(_U):
            i = base + r0 + u
            lo = lo_ref[i]
            los.append(lo)
            nws.append(nw_ref[i])
            xs = jnp.minimum(lo, _N - _WP)
            d2, m, xw = pair_block(i, xs, None)
            d2s.append(d2)
            masks.append(m)
            xls.append(xw)
        msg = filt(jnp.concatenate(d2s, axis=0),
                   jnp.concatenate(masks, axis=0),
                   jnp.concatenate(xls, axis=0))
        xl_rows = xl_ref[pl.ds(base + r0, _U), :]
        out_ref[pl.ds(r0, _U), :] = jnp.dot(
            sel, msg, preferred_element_type=jnp.float32) - xl_rows * wt0

        nw_max = functools.reduce(jnp.maximum, nws)

        @pl.when(nw_max > 1)
        def _rest():
            for u in range(_U):
                i = base + r0 + u
                lo = los[u]

                def win_body(k, acc, i=i, lo=lo):
                    uc = lo + k * _WP
                    xs = jnp.minimum(uc, _N - _WP)
                    d2, m, xw = pair_block(i, xs, uc)
                    msg1 = filt(d2, m, xw)
                    return acc + jnp.sum(msg1, axis=0, keepdims=True)

                extra = jax.lax.fori_loop(
                    1, nws[u], win_body, jnp.zeros((1, _HIDDEN), jnp.float32))
                out_ref[pl.ds(r0 + u, 1), :] += extra

    def body(gg, carry):
        for s in range(_G2):
            group(gg * _G2 + s)
        return carry

    jax.lax.fori_loop(0, _BLK // (_U * _G2), body, 0)


def _post_kernel(agg_ref, h_ref, w2c_ref, b2c_ref, lw_ref, lb_ref, w1n_ref,
                 hout_ref, xlout_ref):
    xc = jnp.dot(agg_ref[...], w2c_ref[...],
                 preferred_element_type=jnp.float32) + b2c_ref[...]
    xo = jnp.dot(_ssp(xc), lw_ref[...],
                 preferred_element_type=jnp.float32) + lb_ref[...]
    h = h_ref[...] + xo
    hout_ref[...] = h
    xlout_ref[...] = jnp.dot(h, w1n_ref[...], preferred_element_type=jnp.float32)


def _head_kernel(h_ref, l1w_ref, l1b_ref, l2w_ref, l2b_ref, batchT_ref, out_ref):
    j = pl.program_id(0)
    t = _ssp(jnp.dot(h_ref[...], l1w_ref[...],
                     preferred_element_type=jnp.float32) + l1b_ref[...])
    y = jnp.dot(t, l2w_ref[...], preferred_element_type=jnp.float32) + l2b_ref[...]
    g = jax.lax.broadcasted_iota(jnp.int32, (_NG, 1), 0)
    onehot_t = (batchT_ref[...] == g).astype(jnp.float32)

    @pl.when(j == 0)
    def _():
        out_ref[...] = jnp.zeros_like(out_ref)

    out_ref[...] += jnp.dot(onehot_t, y, preferred_element_type=jnp.float32)


def _full(shape):
    return pl.BlockSpec(shape, lambda b: tuple(0 for _ in shape))


def kernel(z, pos, batch, emb, mlp_w1, mlp_b1, mlp_w2, mlp_b2, cf_w1, cf_w2,
           cf_b2, lin_w, lin_b, lin1_w, lin1_b, lin2_w, lin2_b):
    batch = batch.astype(jnp.int32)
    z2 = z.astype(jnp.int32).reshape(_N, 1)
    batch2 = batch.reshape(_N, 1)
    batch_t = batch.reshape(1, _N)

    gids = jnp.arange(_NG, dtype=jnp.int32)
    seg_lo = jnp.searchsorted(batch, gids, side='left').astype(jnp.int32)
    seg_hi = jnp.searchsorted(batch, gids, side='right').astype(jnp.int32)
    node_lo = seg_lo[batch]
    node_len = seg_hi[batch] - node_lo
    node_nw = (node_len + _WP - 1) // _WP

    nb = _N // _BLK
    nd = _N // _DB

    h, xl = pl.pallas_call(
        _embed_kernel,
        grid=(nd,),
        in_specs=[
            pl.BlockSpec((_DB, 1), lambda j: (j, 0)),
            _full((100, _HIDDEN)),
            _full((_HIDDEN, _FILTERS)),
        ],
        out_specs=[
            pl.BlockSpec((_DB, _HIDDEN), lambda j: (j, 0)),
            pl.BlockSpec((_DB, _FILTERS), lambda j: (j, 0)),
        ],
        out_shape=[
            jax.ShapeDtypeStruct((_N, _HIDDEN), jnp.float32),
            jax.ShapeDtypeStruct((_N, _FILTERS), jnp.float32),
        ],
    )(z2, emb, cf_w1[0])

    msg_call = pl.pallas_call(
        _msg_kernel,
        grid=(nb,),
        in_specs=[
            pl.BlockSpec(memory_space=pltpu.SMEM),
            pl.BlockSpec(memory_space=pltpu.SMEM),
            _full((_N, 3)),
            _full((_N, 1)),
            _full((_N, _FILTERS)),
            _full((_NUM_G, _FILTERS)),
            _full((1, _FILTERS)),
            _full((_FILTERS, _FILTERS)),
            _full((1, _FILTERS)),
        ],
        out_specs=pl.BlockSpec((_BLK, _FILTERS), lambda b: (b, 0)),
        out_shape=jax.ShapeDtypeStruct((_N, _FILTERS), jnp.float32),
    )

    post_call = pl.pallas_call(
        _post_kernel,
        grid=(nd,),
        in_specs=[
            pl.BlockSpec((_DB, _FILTERS), lambda j: (j, 0)),
            pl.BlockSpec((_DB, _HIDDEN), lambda j: (j, 0)),
            _full((_FILTERS, _HIDDEN)),
            _full((1, _HIDDEN)),
            _full((_HIDDEN, _HIDDEN)),
            _full((1, _HIDDEN)),
            _full((_HIDDEN, _FILTERS)),
        ],
        out_specs=[
            pl.BlockSpec((_DB, _HIDDEN), lambda j: (j, 0)),
            pl.BlockSpec((_DB, _FILTERS), lambda j: (j, 0)),
        ],
        out_shape=[
            jax.ShapeDtypeStruct((_N, _HIDDEN), jnp.float32),
            jax.ShapeDtypeStruct((_N, _FILTERS), jnp.float32),
        ],
    )

    for i in range(_NUM_INT):
        agg = msg_call(node_lo, node_nw, pos, batch2, xl,
                       mlp_w1[i], mlp_b1[i].reshape(1, _FILTERS),
                       mlp_w2[i], mlp_b2[i].reshape(1, _FILTERS))
        w1n = cf_w1[(i + 1) % _NUM_INT]
        h, xl = post_call(agg, h, cf_w2[i], cf_b2[i].reshape(1, _HIDDEN),
                          lin_w[i], lin_b[i].reshape(1, _HIDDEN), w1n)

    out = pl.pallas_call(
        _head_kernel,
        grid=(nd,),
        in_specs=[
            pl.BlockSpec((_DB, _HIDDEN), lambda j: (j, 0)),
            _full((_HIDDEN, _HIDDEN // 2)),
            _full((1, _HIDDEN // 2)),
            _full((_HIDDEN // 2, 1)),
            _full((1, 1)),
            pl.BlockSpec((1, _DB), lambda j: (0, j)),
        ],
        out_specs=pl.BlockSpec((_NG, 1), lambda j: (0, 0)),
        out_shape=jax.ShapeDtypeStruct((_NG, 1), jnp.float32),
    )(h, lin1_w, lin1_b.reshape(1, _HIDDEN // 2),
      lin2_w, lin2_b.reshape(1, 1), batch_t)

    return out


# speedup vs baseline: 54.5813x; 1.0001x over previous
import functools
import math

import jax
import jax.numpy as jnp
import numpy as np
from jax.experimental import pallas as pl
from jax.experimental.pallas import tpu as pltpu

_HIDDEN = 128
_FILTERS = 128
_NUM_INT = 6
_NUM_G = 50
_CUTOFF = 10.0
_N = 8192
_NG = 1024

_BLK = 256
_WP = 32
_U = 8
_G2 = 2
_DB = 512

_LOG2 = math.log(2.0)
_OFFSET = np.linspace(0.0, _CUTOFF, _NUM_G).astype(np.float32)
_COEFF = float(-0.5 / (_OFFSET[1] - _OFFSET[0]) ** 2)


def _ssp(x):
    return jax.nn.softplus(x) - _LOG2


def _embed_kernel(z_ref, emb_ref, w1_ref, h_ref, xl_ref):
    zb = z_ref[...]
    ids = jax.lax.broadcasted_iota(jnp.int32, (1, 100), 1)
    onehot = (zb == ids).astype(jnp.float32)
    h = jnp.dot(onehot, emb_ref[...], preferred_element_type=jnp.float32)
    h_ref[...] = h
    xl_ref[...] = jnp.dot(h, w1_ref[...], preferred_element_type=jnp.float32)


def _msg_kernel(lo_ref, nw_ref, pos_ref, batch_ref, xl_ref,
                w1_ref, b1_ref, w2_ref, b2_ref, out_ref):
    b = pl.program_id(0)
    base = b * _BLK
    offset = jax.lax.broadcasted_iota(jnp.int32, (1, _NUM_G), 1).astype(
        jnp.float32) * (_CUTOFF / (_NUM_G - 1))
    w1 = w1_ref[...]
    b1 = b1_ref[...]
    w2 = w2_ref[...]
    b2 = b2_ref[...]

    def pair_block(i, xs, uc):
        pos_w = pos_ref[pl.ds(xs, _WP), :]
        p_r = pos_ref[pl.ds(i, 1), :]
        bat_w = batch_ref[pl.ds(xs, _WP), :]
        bat_r = batch_ref[pl.ds(i, 1), :]
        diff = pos_w - p_r
        d2 = jnp.sum(diff * diff, axis=1, keepdims=True)
        mask = (d2 < _CUTOFF * _CUTOFF) & (bat_w == bat_r)
        if uc is not None:
            jidx = xs + jax.lax.broadcasted_iota(jnp.int32, (_WP, 1), 0)
            mask = mask & (jidx >= uc)
        return d2, mask, xl_ref[pl.ds(xs, _WP), :]

    sel = (jax.lax.broadcasted_iota(jnp.int32, (_U, _U * _WP), 1) // _WP ==
           jax.lax.broadcasted_iota(jnp.int32, (_U, _U * _WP), 0)
           ).astype(jnp.float32)

    ea0 = jnp.exp(_COEFF * offset ** 2)
    t0 = _ssp(jnp.dot(ea0, w1, preferred_element_type=jnp.float32) + b1)
    wt0 = jnp.dot(t0, w2, preferred_element_type=jnp.float32) + b2

    def filt(d2, mask, xl_w):
        ew = jnp.sqrt(d2)
        ea = jnp.exp(_COEFF * (ew - offset) ** 2)
        t = _ssp(jnp.dot(ea, w1, preferred_element_type=jnp.float32) + b1)
        cm = jnp.where(mask, 0.5 * (jnp.cos(ew * (math.pi / _CUTOFF)) + 1.0),
                       0.0)
        wtm = jnp.dot(t * cm, w2, preferred_element_type=jnp.float32) + b2 * cm
        return xl_w * wtm

    def group(g):
        r0 = g * _U
        d2s, masks, xls, nws, los = [], [], [], [], []
        for u in range(_U):
            i = base + r0 + u
            lo = lo_ref[i]
            los.append(lo)
            nws.append(nw_ref[i])
            xs = jnp.minimum(lo, _N - _WP)
            d2, m, xw = pair_block(i, xs, None)
            d2s.append(d2)
            masks.append(m)
            xls.append(xw)
        msg = filt(jnp.concatenate(d2s, axis=0),
                   jnp.concatenate(masks, axis=0),
                   jnp.concatenate(xls, axis=0))
        xl_rows = xl_ref[pl.ds(base + r0, _U), :]
        out_ref[pl.ds(r0, _U), :] = jnp.dot(
            sel, msg, preferred_element_type=jnp.float32) - xl_rows * wt0

        nw_max = functools.reduce(jnp.maximum, nws)

        @pl.when(nw_max > 1)
        def _rest():
            for u in range(_U):
                i = base + r0 + u
                lo = los[u]

                def win_body(k, acc, i=i, lo=lo):
                    uc = lo + k * _WP
                    xs = jnp.minimum(uc, _N - _WP)
                    d2, m, xw = pair_block(i, xs, uc)
                    msg1 = filt(d2, m, xw)
                    return acc + jnp.sum(msg1, axis=0, keepdims=True)

                extra = jax.lax.fori_loop(
                    1, nws[u], win_body, jnp.zeros((1, _HIDDEN), jnp.float32))
                out_ref[pl.ds(r0 + u, 1), :] += extra

    def body(gg, carry):
        for s in range(_G2):
            group(gg * _G2 + s)
        return carry

    jax.lax.fori_loop(0, _BLK // (_U * _G2), body, 0)


def _post_kernel(agg_ref, h_ref, w2c_ref, b2c_ref, lw_ref, lb_ref, w1n_ref,
                 hout_ref, xlout_ref):
    xc = jnp.dot(agg_ref[...], w2c_ref[...],
                 preferred_element_type=jnp.float32) + b2c_ref[...]
    xo = jnp.dot(_ssp(xc), lw_ref[...],
                 preferred_element_type=jnp.float32) + lb_ref[...]
    h = h_ref[...] + xo
    hout_ref[...] = h
    xlout_ref[...] = jnp.dot(h, w1n_ref[...], preferred_element_type=jnp.float32)


def _head_kernel(h_ref, l1w_ref, l1b_ref, l2w_ref, l2b_ref, batchT_ref, out_ref):
    j = pl.program_id(0)
    t = _ssp(jnp.dot(h_ref[...], l1w_ref[...],
                     preferred_element_type=jnp.float32) + l1b_ref[...])
    y = jnp.dot(t, l2w_ref[...], preferred_element_type=jnp.float32) + l2b_ref[...]
    g = jax.lax.broadcasted_iota(jnp.int32, (_NG, 1), 0)
    onehot_t = (batchT_ref[...] == g).astype(jnp.float32)

    @pl.when(j == 0)
    def _():
        out_ref[...] = jnp.zeros_like(out_ref)

    out_ref[...] += jnp.dot(onehot_t, y, preferred_element_type=jnp.float32)


def _full(shape):
    return pl.BlockSpec(shape, lambda b: tuple(0 for _ in shape))


_PAR = pltpu.CompilerParams(dimension_semantics=("parallel",))


def kernel(z, pos, batch, emb, mlp_w1, mlp_b1, mlp_w2, mlp_b2, cf_w1, cf_w2,
           cf_b2, lin_w, lin_b, lin1_w, lin1_b, lin2_w, lin2_b):
    batch = batch.astype(jnp.int32)
    z2 = z.astype(jnp.int32).reshape(_N, 1)
    batch2 = batch.reshape(_N, 1)
    batch_t = batch.reshape(1, _N)

    gids = jnp.arange(_NG, dtype=jnp.int32)
    seg_lo = jnp.searchsorted(batch, gids, side='left').astype(jnp.int32)
    seg_hi = jnp.searchsorted(batch, gids, side='right').astype(jnp.int32)
    node_lo = seg_lo[batch]
    node_len = seg_hi[batch] - node_lo
    node_nw = (node_len + _WP - 1) // _WP

    nb = _N // _BLK
    nd = _N // _DB

    h, xl = pl.pallas_call(
        _embed_kernel,
        grid=(nd,),
        in_specs=[
            pl.BlockSpec((_DB, 1), lambda j: (j, 0)),
            _full((100, _HIDDEN)),
            _full((_HIDDEN, _FILTERS)),
        ],
        out_specs=[
            pl.BlockSpec((_DB, _HIDDEN), lambda j: (j, 0)),
            pl.BlockSpec((_DB, _FILTERS), lambda j: (j, 0)),
        ],
        out_shape=[
            jax.ShapeDtypeStruct((_N, _HIDDEN), jnp.float32),
            jax.ShapeDtypeStruct((_N, _FILTERS), jnp.float32),
        ],
        compiler_params=_PAR,
    )(z2, emb, cf_w1[0])

    msg_call = pl.pallas_call(
        _msg_kernel,
        grid=(nb,),
        in_specs=[
            pl.BlockSpec(memory_space=pltpu.SMEM),
            pl.BlockSpec(memory_space=pltpu.SMEM),
            _full((_N, 3)),
            _full((_N, 1)),
            _full((_N, _FILTERS)),
            _full((_NUM_G, _FILTERS)),
            _full((1, _FILTERS)),
            _full((_FILTERS, _FILTERS)),
            _full((1, _FILTERS)),
        ],
        out_specs=pl.BlockSpec((_BLK, _FILTERS), lambda b: (b, 0)),
        out_shape=jax.ShapeDtypeStruct((_N, _FILTERS), jnp.float32),
        compiler_params=_PAR,
    )

    post_call = pl.pallas_call(
        _post_kernel,
        grid=(nd,),
        in_specs=[
            pl.BlockSpec((_DB, _FILTERS), lambda j: (j, 0)),
            pl.BlockSpec((_DB, _HIDDEN), lambda j: (j, 0)),
            _full((_FILTERS, _HIDDEN)),
            _full((1, _HIDDEN)),
            _full((_HIDDEN, _HIDDEN)),
            _full((1, _HIDDEN)),
            _full((_HIDDEN, _FILTERS)),
        ],
        out_specs=[
            pl.BlockSpec((_DB, _HIDDEN), lambda j: (j, 0)),
            pl.BlockSpec((_DB, _FILTERS), lambda j: (j, 0)),
        ],
        out_shape=[
            jax.ShapeDtypeStruct((_N, _HIDDEN), jnp.float32),
            jax.ShapeDtypeStruct((_N, _FILTERS), jnp.float32),
        ],
        compiler_params=_PAR,
    )

    for i in range(_NUM_INT):
        agg = msg_call(node_lo, node_nw, pos, batch2, xl,
                       mlp_w1[i], mlp_b1[i].reshape(1, _FILTERS),
                       mlp_w2[i], mlp_b2[i].reshape(1, _FILTERS))
        w1n = cf_w1[(i + 1) % _NUM_INT]
        h, xl = post_call(agg, h, cf_w2[i], cf_b2[i].reshape(1, _HIDDEN),
                          lin_w[i], lin_b[i].reshape(1, _HIDDEN), w1n)

    out = pl.pallas_call(
        _head_kernel,
        grid=(nd,),
        in_specs=[
            pl.BlockSpec((_DB, _HIDDEN), lambda j: (j, 0)),
            _full((_HIDDEN, _HIDDEN // 2)),
            _full((1, _HIDDEN // 2)),
            _full((_HIDDEN // 2, 1)),
            _full((1, 1)),
            pl.BlockSpec((1, _DB), lambda j: (0, j)),
        ],
        out_specs=pl.BlockSpec((_NG, 1), lambda j: (0, 0)),
        out_shape=jax.ShapeDtypeStruct((_NG, 1), jnp.float32),
    )(h, lin1_w, lin1_b.reshape(1, _HIDDEN // 2),
      lin2_w, lin2_b.reshape(1, 1), batch_t)

    return out


# speedup vs baseline: 92.3513x; 1.6920x over previous
import functools
import math

import jax
import jax.numpy as jnp
import numpy as np
from jax.experimental import pallas as pl
from jax.experimental.pallas import tpu as pltpu

_HIDDEN = 128
_FILTERS = 128
_NUM_INT = 6
_NUM_G = 50
_CUTOFF = 10.0
_N = 8192
_NG = 1024

_BLK = 256
_WP = 16
_U = 16
_G2 = 2
_DB = 512

_LOG2 = math.log(2.0)
_OFFSET = np.linspace(0.0, _CUTOFF, _NUM_G).astype(np.float32)
_COEFF = float(-0.5 / (_OFFSET[1] - _OFFSET[0]) ** 2)


def _ssp(x):
    return jax.nn.softplus(x) - _LOG2


def _embed_kernel(z_ref, emb_ref, w1_ref, h_ref, xl_ref):
    zb = z_ref[...]
    ids = jax.lax.broadcasted_iota(jnp.int32, (1, 100), 1)
    onehot = (zb == ids).astype(jnp.float32)
    h = jnp.dot(onehot, emb_ref[...], preferred_element_type=jnp.float32)
    h_ref[...] = h
    xl_ref[...] = jnp.dot(h, w1_ref[...], preferred_element_type=jnp.float32)


def _msg_kernel(lo_ref, nw_ref, pos_ref, batch_ref, xl_ref,
                w1_ref, b1_ref, w2_ref, b2_ref, out_ref):
    b = pl.program_id(0)
    base = b * _BLK
    offset = jax.lax.broadcasted_iota(jnp.int32, (1, _NUM_G), 1).astype(
        jnp.float32) * (_CUTOFF / (_NUM_G - 1))
    w1 = w1_ref[...]
    b1 = b1_ref[...]
    w2 = w2_ref[...]
    b2 = b2_ref[...]

    def pair_block(i, xs, uc):
        pos_w = pos_ref[pl.ds(xs, _WP), :]
        p_r = pos_ref[pl.ds(i, 1), :]
        bat_w = batch_ref[pl.ds(xs, _WP), :]
        bat_r = batch_ref[pl.ds(i, 1), :]
        diff = pos_w - p_r
        d2 = jnp.sum(diff * diff, axis=1, keepdims=True)
        mask = (d2 < _CUTOFF * _CUTOFF) & (bat_w == bat_r)
        if uc is not None:
            jidx = xs + jax.lax.broadcasted_iota(jnp.int32, (_WP, 1), 0)
            mask = mask & (jidx >= uc)
        return d2, mask, xl_ref[pl.ds(xs, _WP), :]

    sel = (jax.lax.broadcasted_iota(jnp.int32, (_U, _U * _WP), 1) // _WP ==
           jax.lax.broadcasted_iota(jnp.int32, (_U, _U * _WP), 0)
           ).astype(jnp.float32)

    ea0 = jnp.exp(_COEFF * offset ** 2)
    t0 = _ssp(jnp.dot(ea0, w1, preferred_element_type=jnp.float32) + b1)
    wt0 = jnp.dot(t0, w2, preferred_element_type=jnp.float32) + b2

    def filt(d2, mask, xl_w):
        ew = jnp.sqrt(d2)
        ea = jnp.exp(_COEFF * (ew - offset) ** 2)
        t = _ssp(jnp.dot(ea, w1, preferred_element_type=jnp.float32) + b1)
        cm = jnp.where(mask, 0.5 * (jnp.cos(ew * (math.pi / _CUTOFF)) + 1.0),
                       0.0)
        wtm = jnp.dot(t * cm, w2, preferred_element_type=jnp.float32) + b2 * cm
        return xl_w * wtm

    def group(g):
        r0 = g * _U
        d2s, masks, xls, nws, los = [], [], [], [], []
        for u in range(_U):
            i = base + r0 + u
            lo = lo_ref[i]
            los.append(lo)
            nws.append(nw_ref[i])
            xs = jnp.minimum(lo, _N - _WP)
            d2, m, xw = pair_block(i, xs, None)
            d2s.append(d2)
            masks.append(m)
            xls.append(xw)
        msg = filt(jnp.concatenate(d2s, axis=0),
                   jnp.concatenate(masks, axis=0),
                   jnp.concatenate(xls, axis=0))
        xl_rows = xl_ref[pl.ds(base + r0, _U), :]
        out_ref[pl.ds(r0, _U), :] = jnp.dot(
            sel, msg, preferred_element_type=jnp.float32) - xl_rows * wt0

        nw_max = functools.reduce(jnp.maximum, nws)

        @pl.when(nw_max > 1)
        def _rest():
            def row_rest(u, carry):
                i = base + r0 + u
                lo = lo_ref[i]

                def win_body(k, acc):
                    uc = lo + k * _WP
                    xs = jnp.minimum(uc, _N - _WP)
                    d2, m, xw = pair_block(i, xs, uc)
                    msg1 = filt(d2, m, xw)
                    return acc + jnp.sum(msg1, axis=0, keepdims=True)

                extra = jax.lax.fori_loop(
                    1, nw_ref[i], win_body,
                    jnp.zeros((1, _HIDDEN), jnp.float32))
                out_ref[pl.ds(r0 + u, 1), :] += extra
                return carry

            jax.lax.fori_loop(0, _U, row_rest, 0)

    def body(gg, carry):
        for s in range(_G2):
            group(gg * _G2 + s)
        return carry

    jax.lax.fori_loop(0, _BLK // (_U * _G2), body, 0)


def _post_kernel(agg_ref, h_ref, w2c_ref, b2c_ref, lw_ref, lb_ref, w1n_ref,
                 hout_ref, xlout_ref):
    xc = jnp.dot(agg_ref[...], w2c_ref[...],
                 preferred_element_type=jnp.float32) + b2c_ref[...]
    xo = jnp.dot(_ssp(xc), lw_ref[...],
                 preferred_element_type=jnp.float32) + lb_ref[...]
    h = h_ref[...] + xo
    hout_ref[...] = h
    xlout_ref[...] = jnp.dot(h, w1n_ref[...], preferred_element_type=jnp.float32)


def _head_kernel(h_ref, l1w_ref, l1b_ref, l2w_ref, l2b_ref, batchT_ref, out_ref):
    j = pl.program_id(0)
    t = _ssp(jnp.dot(h_ref[...], l1w_ref[...],
                     preferred_element_type=jnp.float32) + l1b_ref[...])
    y = jnp.dot(t, l2w_ref[...], preferred_element_type=jnp.float32) + l2b_ref[...]
    g = jax.lax.broadcasted_iota(jnp.int32, (_NG, 1), 0)
    onehot_t = (batchT_ref[...] == g).astype(jnp.float32)

    @pl.when(j == 0)
    def _():
        out_ref[...] = jnp.zeros_like(out_ref)

    out_ref[...] += jnp.dot(onehot_t, y, preferred_element_type=jnp.float32)


def _full(shape):
    return pl.BlockSpec(shape, lambda b: tuple(0 for _ in shape))


_PAR = pltpu.CompilerParams(dimension_semantics=("parallel",))


def kernel(z, pos, batch, emb, mlp_w1, mlp_b1, mlp_w2, mlp_b2, cf_w1, cf_w2,
           cf_b2, lin_w, lin_b, lin1_w, lin1_b, lin2_w, lin2_b):
    batch = batch.astype(jnp.int32)
    z2 = z.astype(jnp.int32).reshape(_N, 1)
    batch2 = batch.reshape(_N, 1)
    batch_t = batch.reshape(1, _N)

    gids = jnp.arange(_NG, dtype=jnp.int32)
    seg_lo = jnp.searchsorted(batch, gids, side='left').astype(jnp.int32)
    seg_hi = jnp.searchsorted(batch, gids, side='right').astype(jnp.int32)
    node_lo = seg_lo[batch]
    node_len = seg_hi[batch] - node_lo
    node_nw = (node_len + _WP - 1) // _WP

    nb = _N // _BLK
    nd = _N // _DB

    h, xl = pl.pallas_call(
        _embed_kernel,
        grid=(nd,),
        in_specs=[
            pl.BlockSpec((_DB, 1), lambda j: (j, 0)),
            _full((100, _HIDDEN)),
            _full((_HIDDEN, _FILTERS)),
        ],
        out_specs=[
            pl.BlockSpec((_DB, _HIDDEN), lambda j: (j, 0)),
            pl.BlockSpec((_DB, _FILTERS), lambda j: (j, 0)),
        ],
        out_shape=[
            jax.ShapeDtypeStruct((_N, _HIDDEN), jnp.float32),
            jax.ShapeDtypeStruct((_N, _FILTERS), jnp.float32),
        ],
        compiler_params=_PAR,
    )(z2, emb, cf_w1[0])

    msg_call = pl.pallas_call(
        _msg_kernel,
        grid=(nb,),
        in_specs=[
            pl.BlockSpec(memory_space=pltpu.SMEM),
            pl.BlockSpec(memory_space=pltpu.SMEM),
            _full((_N, 3)),
            _full((_N, 1)),
            _full((_N, _FILTERS)),
            _full((_NUM_G, _FILTERS)),
            _full((1, _FILTERS)),
            _full((_FILTERS, _FILTERS)),
            _full((1, _FILTERS)),
        ],
        out_specs=pl.BlockSpec((_BLK, _FILTERS), lambda b: (b, 0)),
        out_shape=jax.ShapeDtypeStruct((_N, _FILTERS), jnp.float32),
        compiler_params=_PAR,
    )

    post_call = pl.pallas_call(
        _post_kernel,
        grid=(nd,),
        in_specs=[
            pl.BlockSpec((_DB, _FILTERS), lambda j: (j, 0)),
            pl.BlockSpec((_DB, _HIDDEN), lambda j: (j, 0)),
            _full((_FILTERS, _HIDDEN)),
            _full((1, _HIDDEN)),
            _full((_HIDDEN, _HIDDEN)),
            _full((1, _HIDDEN)),
            _full((_HIDDEN, _FILTERS)),
        ],
        out_specs=[
            pl.BlockSpec((_DB, _HIDDEN), lambda j: (j, 0)),
            pl.BlockSpec((_DB, _FILTERS), lambda j: (j, 0)),
        ],
        out_shape=[
            jax.ShapeDtypeStruct((_N, _HIDDEN), jnp.float32),
            jax.ShapeDtypeStruct((_N, _FILTERS), jnp.float32),
        ],
        compiler_params=_PAR,
    )

    for i in range(_NUM_INT):
        agg = msg_call(node_lo, node_nw, pos, batch2, xl,
                       mlp_w1[i], mlp_b1[i].reshape(1, _FILTERS),
                       mlp_w2[i], mlp_b2[i].reshape(1, _FILTERS))
        w1n = cf_w1[(i + 1) % _NUM_INT]
        h, xl = post_call(agg, h, cf_w2[i], cf_b2[i].reshape(1, _HIDDEN),
                          lin_w[i], lin_b[i].reshape(1, _HIDDEN), w1n)

    out = pl.pallas_call(
        _head_kernel,
        grid=(nd,),
        in_specs=[
            pl.BlockSpec((_DB, _HIDDEN), lambda j: (j, 0)),
            _full((_HIDDEN, _HIDDEN // 2)),
            _full((1, _HIDDEN // 2)),
            _full((_HIDDEN // 2, 1)),
            _full((1, 1)),
            pl.BlockSpec((1, _DB), lambda j: (0, j)),
        ],
        out_specs=pl.BlockSpec((_NG, 1), lambda j: (0, 0)),
        out_shape=jax.ShapeDtypeStruct((_NG, 1), jnp.float32),
    )(h, lin1_w, lin1_b.reshape(1, _HIDDEN // 2),
      lin2_w, lin2_b.reshape(1, 1), batch_t)

    return out


# speedup vs baseline: 113.2949x; 1.2268x over previous
import functools
import math

import jax
import jax.numpy as jnp
import numpy as np
from jax.experimental import pallas as pl
from jax.experimental.pallas import tpu as pltpu

_HIDDEN = 128
_FILTERS = 128
_NUM_INT = 6
_NUM_G = 50
_CUTOFF = 10.0
_N = 8192
_NG = 1024

_BLK = 256
_WP = 16
_U = 32
_G2 = 2
_DB = 512

_LOG2 = math.log(2.0)
_OFFSET = np.linspace(0.0, _CUTOFF, _NUM_G).astype(np.float32)
_COEFF = float(-0.5 / (_OFFSET[1] - _OFFSET[0]) ** 2)


def _ssp(x):
    return jax.nn.softplus(x) - _LOG2


def _embed_kernel(z_ref, emb_ref, w1_ref, h_ref, xl_ref):
    zb = z_ref[...]
    ids = jax.lax.broadcasted_iota(jnp.int32, (1, 100), 1)
    onehot = (zb == ids).astype(jnp.float32)
    h = jnp.dot(onehot, emb_ref[...], preferred_element_type=jnp.float32)
    h_ref[...] = h
    xl_ref[...] = jnp.dot(h, w1_ref[...], preferred_element_type=jnp.float32)


def _msg_kernel(lo_ref, nw_ref, pos_ref, batch_ref, xl_ref,
                w1_ref, b1_ref, w2_ref, b2_ref, out_ref):
    b = pl.program_id(0)
    base = b * _BLK
    offset = jax.lax.broadcasted_iota(jnp.int32, (1, _NUM_G), 1).astype(
        jnp.float32) * (_CUTOFF / (_NUM_G - 1))
    w1 = w1_ref[...]
    b1 = b1_ref[...]
    w2 = w2_ref[...]
    b2 = b2_ref[...]

    def pair_block(i, xs, uc):
        pos_w = pos_ref[pl.ds(xs, _WP), :]
        p_r = pos_ref[pl.ds(i, 1), :]
        bat_w = batch_ref[pl.ds(xs, _WP), :]
        bat_r = batch_ref[pl.ds(i, 1), :]
        diff = pos_w - p_r
        d2 = jnp.sum(diff * diff, axis=1, keepdims=True)
        mask = (d2 < _CUTOFF * _CUTOFF) & (bat_w == bat_r)
        if uc is not None:
            jidx = xs + jax.lax.broadcasted_iota(jnp.int32, (_WP, 1), 0)
            mask = mask & (jidx >= uc)
        return d2, mask, xl_ref[pl.ds(xs, _WP), :]

    sel = (jax.lax.broadcasted_iota(jnp.int32, (_U, _U * _WP), 1) // _WP ==
           jax.lax.broadcasted_iota(jnp.int32, (_U, _U * _WP), 0)
           ).astype(jnp.float32)

    ea0 = jnp.exp(_COEFF * offset ** 2)
    t0 = _ssp(jnp.dot(ea0, w1, preferred_element_type=jnp.float32) + b1)
    wt0 = jnp.dot(t0, w2, preferred_element_type=jnp.float32) + b2

    def filt(d2, mask, xl_w):
        ew = jnp.sqrt(d2)
        ea = jnp.exp(_COEFF * (ew - offset) ** 2)
        t = _ssp(jnp.dot(ea, w1, preferred_element_type=jnp.float32) + b1)
        cm = jnp.where(mask, 0.5 * (jnp.cos(ew * (math.pi / _CUTOFF)) + 1.0),
                       0.0)
        wtm = jnp.dot(t * cm, w2, preferred_element_type=jnp.float32) + b2 * cm
        return xl_w * wtm

    def group(g):
        r0 = g * _U
        d2s, masks, xls, nws, los = [], [], [], [], []
        for u in range(_U):
            i = base + r0 + u
            lo = lo_ref[i]
            los.append(lo)
            nws.append(nw_ref[i])
            xs = jnp.minimum(lo, _N - _WP)
            d2, m, xw = pair_block(i, xs, None)
            d2s.append(d2)
            masks.append(m)
            xls.append(xw)
        msg = filt(jnp.concatenate(d2s, axis=0),
                   jnp.concatenate(masks, axis=0),
                   jnp.concatenate(xls, axis=0))
        xl_rows = xl_ref[pl.ds(base + r0, _U), :]
        out_ref[pl.ds(r0, _U), :] = jnp.dot(
            sel, msg, preferred_element_type=jnp.float32) - xl_rows * wt0

        nw_max = functools.reduce(jnp.maximum, nws)

        @pl.when(nw_max > 1)
        def _rest():
            def row_rest(u, carry):
                i = base + r0 + u
                lo = lo_ref[i]

                def win_body(k, acc):
                    uc = lo + k * _WP
                    xs = jnp.minimum(uc, _N - _WP)
                    d2, m, xw = pair_block(i, xs, uc)
                    msg1 = filt(d2, m, xw)
                    return acc + jnp.sum(msg1, axis=0, keepdims=True)

                extra = jax.lax.fori_loop(
                    1, nw_ref[i], win_body,
                    jnp.zeros((1, _HIDDEN), jnp.float32))
                out_ref[pl.ds(r0 + u, 1), :] += extra
                return carry

            jax.lax.fori_loop(0, _U, row_rest, 0)

    def body(gg, carry):
        for s in range(_G2):
            group(gg * _G2 + s)
        return carry

    jax.lax.fori_loop(0, _BLK // (_U * _G2), body, 0)


def _post_kernel(agg_ref, h_ref, w2c_ref, b2c_ref, lw_ref, lb_ref, w1n_ref,
                 hout_ref, xlout_ref):
    xc = jnp.dot(agg_ref[...], w2c_ref[...],
                 preferred_element_type=jnp.float32) + b2c_ref[...]
    xo = jnp.dot(_ssp(xc), lw_ref[...],
                 preferred_element_type=jnp.float32) + lb_ref[...]
    h = h_ref[...] + xo
    hout_ref[...] = h
    xlout_ref[...] = jnp.dot(h, w1n_ref[...], preferred_element_type=jnp.float32)


def _head_kernel(h_ref, l1w_ref, l1b_ref, l2w_ref, l2b_ref, batchT_ref, out_ref):
    j = pl.program_id(0)
    t = _ssp(jnp.dot(h_ref[...], l1w_ref[...],
                     preferred_element_type=jnp.float32) + l1b_ref[...])
    y = jnp.dot(t, l2w_ref[...], preferred_element_type=jnp.float32) + l2b_ref[...]
    g = jax.lax.broadcasted_iota(jnp.int32, (_NG, 1), 0)
    onehot_t = (batchT_ref[...] == g).astype(jnp.float32)

    @pl.when(j == 0)
    def _():
        out_ref[...] = jnp.zeros_like(out_ref)

    out_ref[...] += jnp.dot(onehot_t, y, preferred_element_type=jnp.float32)


def _full(shape):
    return pl.BlockSpec(shape, lambda b: tuple(0 for _ in shape))


_PAR = pltpu.CompilerParams(dimension_semantics=("parallel",))


def kernel(z, pos, batch, emb, mlp_w1, mlp_b1, mlp_w2, mlp_b2, cf_w1, cf_w2,
           cf_b2, lin_w, lin_b, lin1_w, lin1_b, lin2_w, lin2_b):
    batch = batch.astype(jnp.int32)
    z2 = z.astype(jnp.int32).reshape(_N, 1)
    batch2 = batch.reshape(_N, 1)
    batch_t = batch.reshape(1, _N)

    gids = jnp.arange(_NG, dtype=jnp.int32)
    seg_lo = jnp.searchsorted(batch, gids, side='left').astype(jnp.int32)
    seg_hi = jnp.searchsorted(batch, gids, side='right').astype(jnp.int32)
    node_lo = seg_lo[batch]
    node_len = seg_hi[batch] - node_lo
    node_nw = (node_len + _WP - 1) // _WP

    nb = _N // _BLK
    nd = _N // _DB

    h, xl = pl.pallas_call(
        _embed_kernel,
        grid=(nd,),
        in_specs=[
            pl.BlockSpec((_DB, 1), lambda j: (j, 0)),
            _full((100, _HIDDEN)),
            _full((_HIDDEN, _FILTERS)),
        ],
        out_specs=[
            pl.BlockSpec((_DB, _HIDDEN), lambda j: (j, 0)),
            pl.BlockSpec((_DB, _FILTERS), lambda j: (j, 0)),
        ],
        out_shape=[
            jax.ShapeDtypeStruct((_N, _HIDDEN), jnp.float32),
            jax.ShapeDtypeStruct((_N, _FILTERS), jnp.float32),
        ],
        compiler_params=_PAR,
    )(z2, emb, cf_w1[0])

    msg_call = pl.pallas_call(
        _msg_kernel,
        grid=(nb,),
        in_specs=[
            pl.BlockSpec(memory_space=pltpu.SMEM),
            pl.BlockSpec(memory_space=pltpu.SMEM),
            _full((_N, 3)),
            _full((_N, 1)),
            _full((_N, _FILTERS)),
            _full((_NUM_G, _FILTERS)),
            _full((1, _FILTERS)),
            _full((_FILTERS, _FILTERS)),
            _full((1, _FILTERS)),
        ],
        out_specs=pl.BlockSpec((_BLK, _FILTERS), lambda b: (b, 0)),
        out_shape=jax.ShapeDtypeStruct((_N, _FILTERS), jnp.float32),
        compiler_params=_PAR,
    )

    post_call = pl.pallas_call(
        _post_kernel,
        grid=(nd,),
        in_specs=[
            pl.BlockSpec((_DB, _FILTERS), lambda j: (j, 0)),
            pl.BlockSpec((_DB, _HIDDEN), lambda j: (j, 0)),
            _full((_FILTERS, _HIDDEN)),
            _full((1, _HIDDEN)),
            _full((_HIDDEN, _HIDDEN)),
            _full((1, _HIDDEN)),
            _full((_HIDDEN, _FILTERS)),
        ],
        out_specs=[
            pl.BlockSpec((_DB, _HIDDEN), lambda j: (j, 0)),
            pl.BlockSpec((_DB, _FILTERS), lambda j: (j, 0)),
        ],
        out_shape=[
            jax.ShapeDtypeStruct((_N, _HIDDEN), jnp.float32),
            jax.ShapeDtypeStruct((_N, _FILTERS), jnp.float32),
        ],
        compiler_params=_PAR,
    )

    for i in range(_NUM_INT):
        agg = msg_call(node_lo, node_nw, pos, batch2, xl,
                       mlp_w1[i], mlp_b1[i].reshape(1, _FILTERS),
                       mlp_w2[i], mlp_b2[i].reshape(1, _FILTERS))
        w1n = cf_w1[(i + 1) % _NUM_INT]
        h, xl = post_call(agg, h, cf_w2[i], cf_b2[i].reshape(1, _HIDDEN),
                          lin_w[i], lin_b[i].reshape(1, _HIDDEN), w1n)

    out = pl.pallas_call(
        _head_kernel,
        grid=(nd,),
        in_specs=[
            pl.BlockSpec((_DB, _HIDDEN), lambda j: (j, 0)),
            _full((_HIDDEN, _HIDDEN // 2)),
            _full((1, _HIDDEN // 2)),
            _full((_HIDDEN // 2, 1)),
            _full((1, 1)),
            pl.BlockSpec((1, _DB), lambda j: (0, j)),
        ],
        out_specs=pl.BlockSpec((_NG, 1), lambda j: (0, 0)),
        out_shape=jax.ShapeDtypeStruct((_NG, 1), jnp.float32),
    )(h, lin1_w, lin1_b.reshape(1, _HIDDEN // 2),
      lin2_w, lin2_b.reshape(1, 1), batch_t)

    return out


# speedup vs baseline: 114.7469x; 1.0128x over previous
import functools
import math

import jax
import jax.numpy as jnp
import numpy as np
from jax.experimental import pallas as pl
from jax.experimental.pallas import tpu as pltpu

_HIDDEN = 128
_FILTERS = 128
_NUM_INT = 6
_NUM_G = 50
_CUTOFF = 10.0
_N = 8192
_NG = 1024

_BLK = 256
_WP = 16
_U = 32
_G2 = 2
_DB = 512

_LOG2 = math.log(2.0)
_OFFSET = np.linspace(0.0, _CUTOFF, _NUM_G).astype(np.float32)
_COEFF = float(-0.5 / (_OFFSET[1] - _OFFSET[0]) ** 2)


def _ssp(x):
    return jax.nn.softplus(x) - _LOG2


def _embed_kernel(z_ref, emb_ref, w1_ref, h_ref, xl_ref):
    zb = z_ref[...]
    ids = jax.lax.broadcasted_iota(jnp.int32, (1, 100), 1)
    onehot = (zb == ids).astype(jnp.float32)
    h = jnp.dot(onehot, emb_ref[...], preferred_element_type=jnp.float32)
    h_ref[...] = h
    xl_ref[...] = jnp.dot(h, w1_ref[...], preferred_element_type=jnp.float32)


def _msg_kernel(lo_ref, nw_ref, pos_ref, xl_ref,
                w1_ref, b1_ref, w2_ref, b2_ref, out_ref):
    b = pl.program_id(0)
    base = b * _BLK
    offset = jax.lax.broadcasted_iota(jnp.int32, (1, _NUM_G), 1).astype(
        jnp.float32) * (_CUTOFF / (_NUM_G - 1))
    w1 = w1_ref[...]
    b1 = b1_ref[...]
    w2 = w2_ref[...]
    b2 = b2_ref[...]

    def pair_block(i, xs, uc):
        pos_w = pos_ref[pl.ds(xs, _WP), :]
        p_r = pos_ref[pl.ds(i, 1), :]
        diff = pos_w - p_r
        d2 = jnp.sum(diff * diff, axis=1, keepdims=True)
        if uc is not None:
            jidx = xs + jax.lax.broadcasted_iota(jnp.int32, (_WP, 1), 0)
            d2 = jnp.where(jidx >= uc, d2, 1e9)
        return d2, xl_ref[pl.ds(xs, _WP), :]

    sel = (jax.lax.broadcasted_iota(jnp.int32, (_U, _U * _WP), 1) // _WP ==
           jax.lax.broadcasted_iota(jnp.int32, (_U, _U * _WP), 0)
           ).astype(jnp.float32)

    ea0 = jnp.exp(_COEFF * offset ** 2)
    t0 = _ssp(jnp.dot(ea0, w1, preferred_element_type=jnp.float32) + b1)
    wt0 = jnp.dot(t0, w2, preferred_element_type=jnp.float32) + b2

    def filt(d2, xl_w):
        ew = jnp.sqrt(d2)
        ea = jnp.exp(_COEFF * (ew - offset) ** 2)
        t = _ssp(jnp.dot(ea, w1, preferred_element_type=jnp.float32) + b1)
        cm = jnp.where(d2 < _CUTOFF * _CUTOFF,
                       0.5 * (jnp.cos(ew * (math.pi / _CUTOFF)) + 1.0),
                       0.0)
        wtm = jnp.dot(t * cm, w2, preferred_element_type=jnp.float32) + b2 * cm
        return xl_w * wtm

    def group(g):
        r0 = g * _U
        d2s, xls, nws = [], [], []
        for u in range(_U):
            i = base + r0 + u
            lo = lo_ref[i]
            nws.append(nw_ref[i])
            xs = jnp.minimum(lo, _N - _WP)
            d2, xw = pair_block(i, xs, None)
            d2s.append(d2)
            xls.append(xw)
        msg = filt(jnp.concatenate(d2s, axis=0),
                   jnp.concatenate(xls, axis=0))
        xl_rows = xl_ref[pl.ds(base + r0, _U), :]
        out_ref[pl.ds(r0, _U), :] = jnp.dot(
            sel, msg, preferred_element_type=jnp.float32) - xl_rows * wt0

        nw_max = functools.reduce(jnp.maximum, nws)

        @pl.when(nw_max > 1)
        def _rest():
            def row_rest(u, carry):
                i = base + r0 + u
                lo = lo_ref[i]

                def win_body(k, acc):
                    uc = lo + k * _WP
                    xs = jnp.minimum(uc, _N - _WP)
                    d2, xw = pair_block(i, xs, uc)
                    msg1 = filt(d2, xw)
                    return acc + jnp.sum(msg1, axis=0, keepdims=True)

                extra = jax.lax.fori_loop(
                    1, nw_ref[i], win_body,
                    jnp.zeros((1, _HIDDEN), jnp.float32))
                out_ref[pl.ds(r0 + u, 1), :] += extra
                return carry

            jax.lax.fori_loop(0, _U, row_rest, 0)

    def body(gg, carry):
        for s in range(_G2):
            group(gg * _G2 + s)
        return carry

    jax.lax.fori_loop(0, _BLK // (_U * _G2), body, 0)


def _post_kernel(agg_ref, h_ref, w2c_ref, b2c_ref, lw_ref, lb_ref, w1n_ref,
                 hout_ref, xlout_ref):
    xc = jnp.dot(agg_ref[...], w2c_ref[...],
                 preferred_element_type=jnp.float32) + b2c_ref[...]
    xo = jnp.dot(_ssp(xc), lw_ref[...],
                 preferred_element_type=jnp.float32) + lb_ref[...]
    h = h_ref[...] + xo
    hout_ref[...] = h
    xlout_ref[...] = jnp.dot(h, w1n_ref[...], preferred_element_type=jnp.float32)


def _head_kernel(h_ref, l1w_ref, l1b_ref, l2w_ref, l2b_ref, batchT_ref, out_ref):
    j = pl.program_id(0)
    t = _ssp(jnp.dot(h_ref[...], l1w_ref[...],
                     preferred_element_type=jnp.float32) + l1b_ref[...])
    y = jnp.dot(t, l2w_ref[...], preferred_element_type=jnp.float32) + l2b_ref[...]
    g = jax.lax.broadcasted_iota(jnp.int32, (_NG, 1), 0)
    onehot_t = (batchT_ref[...] == g).astype(jnp.float32)

    @pl.when(j == 0)
    def _():
        out_ref[...] = jnp.zeros_like(out_ref)

    out_ref[...] += jnp.dot(onehot_t, y, preferred_element_type=jnp.float32)


def _full(shape):
    return pl.BlockSpec(shape, lambda b: tuple(0 for _ in shape))


_PAR = pltpu.CompilerParams(dimension_semantics=("parallel",))


def kernel(z, pos, batch, emb, mlp_w1, mlp_b1, mlp_w2, mlp_b2, cf_w1, cf_w2,
           cf_b2, lin_w, lin_b, lin1_w, lin1_b, lin2_w, lin2_b):
    batch = batch.astype(jnp.int32)
    z2 = z.astype(jnp.int32).reshape(_N, 1)
    batch2 = batch.reshape(_N, 1)
    batch_t = batch.reshape(1, _N)

    gids = jnp.arange(_NG, dtype=jnp.int32)
    seg_lo = jnp.searchsorted(batch, gids, side='left').astype(jnp.int32)
    seg_hi = jnp.searchsorted(batch, gids, side='right').astype(jnp.int32)
    node_lo = seg_lo[batch]
    node_len = seg_hi[batch] - node_lo
    node_nw = (node_len + _WP - 1) // _WP

    pos4 = jnp.concatenate(
        [pos, batch.astype(jnp.float32).reshape(_N, 1) * 1000.0], axis=1)

    nb = _N // _BLK
    nd = _N // _DB

    h, xl = pl.pallas_call(
        _embed_kernel,
        grid=(nd,),
        in_specs=[
            pl.BlockSpec((_DB, 1), lambda j: (j, 0)),
            _full((100, _HIDDEN)),
            _full((_HIDDEN, _FILTERS)),
        ],
        out_specs=[
            pl.BlockSpec((_DB, _HIDDEN), lambda j: (j, 0)),
            pl.BlockSpec((_DB, _FILTERS), lambda j: (j, 0)),
        ],
        out_shape=[
            jax.ShapeDtypeStruct((_N, _HIDDEN), jnp.float32),
            jax.ShapeDtypeStruct((_N, _FILTERS), jnp.float32),
        ],
        compiler_params=_PAR,
    )(z2, emb, cf_w1[0])

    msg_call = pl.pallas_call(
        _msg_kernel,
        grid=(nb,),
        in_specs=[
            pl.BlockSpec(memory_space=pltpu.SMEM),
            pl.BlockSpec(memory_space=pltpu.SMEM),
            _full((_N, 4)),
            _full((_N, _FILTERS)),
            _full((_NUM_G, _FILTERS)),
            _full((1, _FILTERS)),
            _full((_FILTERS, _FILTERS)),
            _full((1, _FILTERS)),
        ],
        out_specs=pl.BlockSpec((_BLK, _FILTERS), lambda b: (b, 0)),
        out_shape=jax.ShapeDtypeStruct((_N, _FILTERS), jnp.float32),
        compiler_params=_PAR,
    )

    post_call = pl.pallas_call(
        _post_kernel,
        grid=(nd,),
        in_specs=[
            pl.BlockSpec((_DB, _FILTERS), lambda j: (j, 0)),
            pl.BlockSpec((_DB, _HIDDEN), lambda j: (j, 0)),
            _full((_FILTERS, _HIDDEN)),
            _full((1, _HIDDEN)),
            _full((_HIDDEN, _HIDDEN)),
            _full((1, _HIDDEN)),
            _full((_HIDDEN, _FILTERS)),
        ],
        out_specs=[
            pl.BlockSpec((_DB, _HIDDEN), lambda j: (j, 0)),
            pl.BlockSpec((_DB, _FILTERS), lambda j: (j, 0)),
        ],
        out_shape=[
            jax.ShapeDtypeStruct((_N, _HIDDEN), jnp.float32),
            jax.ShapeDtypeStruct((_N, _FILTERS), jnp.float32),
        ],
        compiler_params=_PAR,
    )

    for i in range(_NUM_INT):
        agg = msg_call(node_lo, node_nw, pos4, xl,
                       mlp_w1[i], mlp_b1[i].reshape(1, _FILTERS),
                       mlp_w2[i], mlp_b2[i].reshape(1, _FILTERS))
        w1n = cf_w1[(i + 1) % _NUM_INT]
        h, xl = post_call(agg, h, cf_w2[i], cf_b2[i].reshape(1, _HIDDEN),
                          lin_w[i], lin_b[i].reshape(1, _HIDDEN), w1n)

    out = pl.pallas_call(
        _head_kernel,
        grid=(nd,),
        in_specs=[
            pl.BlockSpec((_DB, _HIDDEN), lambda j: (j, 0)),
            _full((_HIDDEN, _HIDDEN // 2)),
            _full((1, _HIDDEN // 2)),
            _full((_HIDDEN // 2, 1)),
            _full((1, 1)),
            pl.BlockSpec((1, _DB), lambda j: (0, j)),
        ],
        out_specs=pl.BlockSpec((_NG, 1), lambda j: (0, 0)),
        out_shape=jax.ShapeDtypeStruct((_NG, 1), jnp.float32),
    )(h, lin1_w, lin1_b.reshape(1, _HIDDEN // 2),
      lin2_w, lin2_b.reshape(1, 1), batch_t)

    return out


# speedup vs baseline: 116.1396x; 1.0121x over previous
import functools
import math

import jax
import jax.numpy as jnp
import numpy as np
from jax.experimental import pallas as pl
from jax.experimental.pallas import tpu as pltpu

_HIDDEN = 128
_FILTERS = 128
_NUM_INT = 6
_NUM_G = 50
_CUTOFF = 10.0
_N = 8192
_NG = 1024

_BLK = 512
_WP = 16
_U = 32
_G2 = 4
_DB = 512

_LOG2 = math.log(2.0)
_OFFSET = np.linspace(0.0, _CUTOFF, _NUM_G).astype(np.float32)
_COEFF = float(-0.5 / (_OFFSET[1] - _OFFSET[0]) ** 2)


def _ssp(x):
    return jax.nn.softplus(x) - _LOG2


def _embed_kernel(z_ref, emb_ref, w1_ref, h_ref, xl_ref):
    zb = z_ref[...]
    ids = jax.lax.broadcasted_iota(jnp.int32, (1, 100), 1)
    onehot = (zb == ids).astype(jnp.float32)
    h = jnp.dot(onehot, emb_ref[...], preferred_element_type=jnp.float32)
    h_ref[...] = h
    xl_ref[...] = jnp.dot(h, w1_ref[...], preferred_element_type=jnp.float32)


def _msg_kernel(lo_ref, nw_ref, pos_ref, xl_ref,
                w1_ref, b1_ref, w2_ref, b2_ref, out_ref):
    b = pl.program_id(0)
    base = b * _BLK
    offset = jax.lax.broadcasted_iota(jnp.int32, (1, _NUM_G), 1).astype(
        jnp.float32) * (_CUTOFF / (_NUM_G - 1))
    w1 = w1_ref[...]
    b1 = b1_ref[...]
    w2 = w2_ref[...]
    b2 = b2_ref[...]

    def pair_block(i, xs, uc):
        pos_w = pos_ref[pl.ds(xs, _WP), :]
        p_r = pos_ref[pl.ds(i, 1), :]
        diff = pos_w - p_r
        d2 = jnp.sum(diff * diff, axis=1, keepdims=True)
        if uc is not None:
            jidx = xs + jax.lax.broadcasted_iota(jnp.int32, (_WP, 1), 0)
            d2 = jnp.where(jidx >= uc, d2, 1e9)
        return d2, xl_ref[pl.ds(xs, _WP), :]

    sel = (jax.lax.broadcasted_iota(jnp.int32, (_U, _U * _WP), 1) // _WP ==
           jax.lax.broadcasted_iota(jnp.int32, (_U, _U * _WP), 0)
           ).astype(jnp.float32)

    ea0 = jnp.exp(_COEFF * offset ** 2)
    t0 = _ssp(jnp.dot(ea0, w1, preferred_element_type=jnp.float32) + b1)
    wt0 = jnp.dot(t0, w2, preferred_element_type=jnp.float32) + b2

    def filt(d2, xl_w):
        ew = jnp.sqrt(d2)
        ea = jnp.exp(_COEFF * (ew - offset) ** 2)
        t = _ssp(jnp.dot(ea, w1, preferred_element_type=jnp.float32) + b1)
        cm = jnp.where(d2 < _CUTOFF * _CUTOFF,
                       0.5 * (jnp.cos(ew * (math.pi / _CUTOFF)) + 1.0),
                       0.0)
        wtm = jnp.dot(t * cm, w2, preferred_element_type=jnp.float32) + b2 * cm
        return xl_w * wtm

    def group(g):
        r0 = g * _U
        d2s, xls, nws = [], [], []
        for u in range(_U):
            i = base + r0 + u
            lo = lo_ref[i]
            nws.append(nw_ref[i])
            xs = jnp.minimum(lo, _N - _WP)
            d2, xw = pair_block(i, xs, None)
            d2s.append(d2)
            xls.append(xw)
        msg = filt(jnp.concatenate(d2s, axis=0),
                   jnp.concatenate(xls, axis=0))
        xl_rows = xl_ref[pl.ds(base + r0, _U), :]
        out_ref[pl.ds(r0, _U), :] = jnp.dot(
            sel, msg, preferred_element_type=jnp.float32) - xl_rows * wt0

        nw_max = functools.reduce(jnp.maximum, nws)

        @pl.when(nw_max > 1)
        def _rest():
            def row_rest(u, carry):
                i = base + r0 + u
                lo = lo_ref[i]

                def win_body(k, acc):
                    uc = lo + k * _WP
                    xs = jnp.minimum(uc, _N - _WP)
                    d2, xw = pair_block(i, xs, uc)
                    msg1 = filt(d2, xw)
                    return acc + jnp.sum(msg1, axis=0, keepdims=True)

                extra = jax.lax.fori_loop(
                    1, nw_ref[i], win_body,
                    jnp.zeros((1, _HIDDEN), jnp.float32))
                out_ref[pl.ds(r0 + u, 1), :] += extra
                return carry

            jax.lax.fori_loop(0, _U, row_rest, 0)

    def body(gg, carry):
        for s in range(_G2):
            group(gg * _G2 + s)
        return carry

    jax.lax.fori_loop(0, _BLK // (_U * _G2), body, 0)


def _post_kernel(agg_ref, h_ref, w2c_ref, b2c_ref, lw_ref, lb_ref, w1n_ref,
                 hout_ref, xlout_ref):
    xc = jnp.dot(agg_ref[...], w2c_ref[...],
                 preferred_element_type=jnp.float32) + b2c_ref[...]
    xo = jnp.dot(_ssp(xc), lw_ref[...],
                 preferred_element_type=jnp.float32) + lb_ref[...]
    h = h_ref[...] + xo
    hout_ref[...] = h
    xlout_ref[...] = jnp.dot(h, w1n_ref[...], preferred_element_type=jnp.float32)


def _head_kernel(h_ref, l1w_ref, l1b_ref, l2w_ref, l2b_ref, batchT_ref, out_ref):
    j = pl.program_id(0)
    t = _ssp(jnp.dot(h_ref[...], l1w_ref[...],
                     preferred_element_type=jnp.float32) + l1b_ref[...])
    y = jnp.dot(t, l2w_ref[...], preferred_element_type=jnp.float32) + l2b_ref[...]
    g = jax.lax.broadcasted_iota(jnp.int32, (_NG, 1), 0)
    onehot_t = (batchT_ref[...] == g).astype(jnp.float32)

    @pl.when(j == 0)
    def _():
        out_ref[...] = jnp.zeros_like(out_ref)

    out_ref[...] += jnp.dot(onehot_t, y, preferred_element_type=jnp.float32)


def _full(shape):
    return pl.BlockSpec(shape, lambda b: tuple(0 for _ in shape))


_PAR = pltpu.CompilerParams(dimension_semantics=("parallel",))


def kernel(z, pos, batch, emb, mlp_w1, mlp_b1, mlp_w2, mlp_b2, cf_w1, cf_w2,
           cf_b2, lin_w, lin_b, lin1_w, lin1_b, lin2_w, lin2_b):
    batch = batch.astype(jnp.int32)
    z2 = z.astype(jnp.int32).reshape(_N, 1)
    batch2 = batch.reshape(_N, 1)
    batch_t = batch.reshape(1, _N)

    gids = jnp.arange(_NG, dtype=jnp.int32)
    seg_lo = jnp.searchsorted(batch, gids, side='left').astype(jnp.int32)
    seg_hi = jnp.searchsorted(batch, gids, side='right').astype(jnp.int32)
    node_lo = seg_lo[batch]
    node_len = seg_hi[batch] - node_lo
    node_nw = (node_len + _WP - 1) // _WP

    pos4 = jnp.concatenate(
        [pos, batch.astype(jnp.float32).reshape(_N, 1) * 1000.0], axis=1)

    nb = _N // _BLK
    nd = _N // _DB

    h, xl = pl.pallas_call(
        _embed_kernel,
        grid=(nd,),
        in_specs=[
            pl.BlockSpec((_DB, 1), lambda j: (j, 0)),
            _full((100, _HIDDEN)),
            _full((_HIDDEN, _FILTERS)),
        ],
        out_specs=[
            pl.BlockSpec((_DB, _HIDDEN), lambda j: (j, 0)),
            pl.BlockSpec((_DB, _FILTERS), lambda j: (j, 0)),
        ],
        out_shape=[
            jax.ShapeDtypeStruct((_N, _HIDDEN), jnp.float32),
            jax.ShapeDtypeStruct((_N, _FILTERS), jnp.float32),
        ],
        compiler_params=_PAR,
    )(z2, emb, cf_w1[0])

    msg_call = pl.pallas_call(
        _msg_kernel,
        grid=(nb,),
        in_specs=[
            pl.BlockSpec(memory_space=pltpu.SMEM),
            pl.BlockSpec(memory_space=pltpu.SMEM),
            _full((_N, 4)),
            _full((_N, _FILTERS)),
            _full((_NUM_G, _FILTERS)),
            _full((1, _FILTERS)),
            _full((_FILTERS, _FILTERS)),
            _full((1, _FILTERS)),
        ],
        out_specs=pl.BlockSpec((_BLK, _FILTERS), lambda b: (b, 0)),
        out_shape=jax.ShapeDtypeStruct((_N, _FILTERS), jnp.float32),
        compiler_params=_PAR,
    )

    post_call = pl.pallas_call(
        _post_kernel,
        grid=(nd,),
        in_specs=[
            pl.BlockSpec((_DB, _FILTERS), lambda j: (j, 0)),
            pl.BlockSpec((_DB, _HIDDEN), lambda j: (j, 0)),
            _full((_FILTERS, _HIDDEN)),
            _full((1, _HIDDEN)),
            _full((_HIDDEN, _HIDDEN)),
            _full((1, _HIDDEN)),
            _full((_HIDDEN, _FILTERS)),
        ],
        out_specs=[
            pl.BlockSpec((_DB, _HIDDEN), lambda j: (j, 0)),
            pl.BlockSpec((_DB, _FILTERS), lambda j: (j, 0)),
        ],
        out_shape=[
            jax.ShapeDtypeStruct((_N, _HIDDEN), jnp.float32),
            jax.ShapeDtypeStruct((_N, _FILTERS), jnp.float32),
        ],
        compiler_params=_PAR,
    )

    for i in range(_NUM_INT):
        agg = msg_call(node_lo, node_nw, pos4, xl,
                       mlp_w1[i], mlp_b1[i].reshape(1, _FILTERS),
                       mlp_w2[i], mlp_b2[i].reshape(1, _FILTERS))
        w1n = cf_w1[(i + 1) % _NUM_INT]
        h, xl = post_call(agg, h, cf_w2[i], cf_b2[i].reshape(1, _HIDDEN),
                          lin_w[i], lin_b[i].reshape(1, _HIDDEN), w1n)

    out = pl.pallas_call(
        _head_kernel,
        grid=(nd,),
        in_specs=[
            pl.BlockSpec((_DB, _HIDDEN), lambda j: (j, 0)),
            _full((_HIDDEN, _HIDDEN // 2)),
            _full((1, _HIDDEN // 2)),
            _full((_HIDDEN // 2, 1)),
            _full((1, 1)),
            pl.BlockSpec((1, _DB), lambda j: (0, j)),
        ],
        out_specs=pl.BlockSpec((_NG, 1), lambda j: (0, 0)),
        out_shape=jax.ShapeDtypeStruct((_NG, 1), jnp.float32),
    )(h, lin1_w, lin1_b.reshape(1, _HIDDEN // 2),
      lin2_w, lin2_b.reshape(1, 1), batch_t)

    return out


# speedup vs baseline: 116.2127x; 1.0006x over previous
import functools
import math

import jax
import jax.numpy as jnp
import numpy as np
from jax.experimental import pallas as pl
from jax.experimental.pallas import tpu as pltpu

_HIDDEN = 128
_FILTERS = 128
_NUM_INT = 6
_NUM_G = 50
_CUTOFF = 10.0
_N = 8192
_NG = 1024

_BLK = 512
_WP = 16
_U = 32
_G2 = 4
_DB = 512

_LOG2 = math.log(2.0)
_OFFSET = np.linspace(0.0, _CUTOFF, _NUM_G).astype(np.float32)
_COEFF = float(-0.5 / (_OFFSET[1] - _OFFSET[0]) ** 2)


def _ssp(x):
    return jax.nn.softplus(x) - _LOG2


def _embed_kernel(z_ref, emb_ref, w1_ref, h_ref, xl_ref):
    zb = z_ref[...]
    ids = jax.lax.broadcasted_iota(jnp.int32, (1, 100), 1)
    onehot = (zb == ids).astype(jnp.float32)
    h = jnp.dot(onehot, emb_ref[...], preferred_element_type=jnp.float32)
    h_ref[...] = h
    xl_ref[...] = jnp.dot(h, w1_ref[...], preferred_element_type=jnp.float32)


def _msg_kernel(lo_ref, nw_ref, pos_ref, xl_ref,
                w1_ref, b1_ref, w2_ref, b2_ref, out_ref):
    b = pl.program_id(0)
    base = b * _BLK
    offset = jax.lax.broadcasted_iota(jnp.int32, (1, _NUM_G), 1).astype(
        jnp.float32) * (_CUTOFF / (_NUM_G - 1))
    w1 = w1_ref[...]
    b1 = b1_ref[...]
    w2 = w2_ref[...]
    b2 = b2_ref[...]

    def pair_block(i, xs, uc):
        pos_w = pos_ref[pl.ds(xs, _WP), :]
        p_r = pos_ref[pl.ds(i, 1), :]
        diff = pos_w - p_r
        d2 = jnp.sum(diff * diff, axis=1, keepdims=True)
        if uc is not None:
            jidx = xs + jax.lax.broadcasted_iota(jnp.int32, (_WP, 1), 0)
            d2 = jnp.where(jidx >= uc, d2, 1e9)
        return d2, xl_ref[pl.ds(xs, _WP), :]

    sel = (jax.lax.broadcasted_iota(jnp.int32, (_U, _U * _WP), 1) // _WP ==
           jax.lax.broadcasted_iota(jnp.int32, (_U, _U * _WP), 0)
           ).astype(jnp.float32)

    ea0 = jnp.exp(_COEFF * offset ** 2)
    t0 = _ssp(jnp.dot(ea0, w1, preferred_element_type=jnp.float32) + b1)
    wt0 = jnp.dot(t0, w2, preferred_element_type=jnp.float32) + b2

    def filt(d2, xl_w):
        ew = jnp.sqrt(d2)
        ea = jnp.exp(_COEFF * (ew - offset) ** 2)
        t = _ssp(jnp.dot(ea, w1, preferred_element_type=jnp.float32) + b1)
        cm = jnp.where(d2 < _CUTOFF * _CUTOFF,
                       0.5 * (jnp.cos(ew * (math.pi / _CUTOFF)) + 1.0),
                       0.0)
        wtm = jnp.dot(t * cm, w2, preferred_element_type=jnp.float32) + b2 * cm
        return xl_w * wtm

    def group(g):
        r0 = g * _U
        d2s, xls, nws = [], [], []
        for u in range(_U):
            i = base + r0 + u
            lo = lo_ref[i]
            nws.append(nw_ref[i])
            xs = jnp.minimum(lo, _N - _WP)
            d2, xw = pair_block(i, xs, None)
            d2s.append(d2)
            xls.append(xw)
        msg = filt(jnp.concatenate(d2s, axis=0),
                   jnp.concatenate(xls, axis=0))
        xl_rows = xl_ref[pl.ds(base + r0, _U), :]
        out_ref[pl.ds(r0, _U), :] = jnp.dot(
            sel, msg, preferred_element_type=jnp.float32) - xl_rows * wt0

        nw_max = functools.reduce(jnp.maximum, nws)

        @pl.when(nw_max > 1)
        def _rest():
            def row_rest(u, carry):
                i = base + r0 + u
                lo = lo_ref[i]

                def win_body(k, acc):
                    uc = lo + k * _WP
                    xs = jnp.minimum(uc, _N - _WP)
                    d2, xw = pair_block(i, xs, uc)
                    msg1 = filt(d2, xw)
                    return acc + jnp.sum(msg1, axis=0, keepdims=True)

                extra = jax.lax.fori_loop(
                    1, nw_ref[i], win_body,
                    jnp.zeros((1, _HIDDEN), jnp.float32))
                out_ref[pl.ds(r0 + u, 1), :] += extra
                return carry

            jax.lax.fori_loop(0, _U, row_rest, 0)

    def body(gg, carry):
        for s in range(_G2):
            group(gg * _G2 + s)
        return carry

    jax.lax.fori_loop(0, _BLK // (_U * _G2), body, 0)


def _post_kernel(agg_ref, h_ref, w2c_ref, b2c_ref, lw_ref, lb_ref, w1n_ref,
                 hout_ref, xlout_ref):
    xc = jnp.dot(agg_ref[...], w2c_ref[...],
                 preferred_element_type=jnp.float32) + b2c_ref[...]
    xo = jnp.dot(_ssp(xc), lw_ref[...],
                 preferred_element_type=jnp.float32) + lb_ref[...]
    h = h_ref[...] + xo
    hout_ref[...] = h
    xlout_ref[...] = jnp.dot(h, w1n_ref[...], preferred_element_type=jnp.float32)


def _head_kernel(h_ref, l1w_ref, l1b_ref, l2w_ref, l2b_ref, batchT_ref, out_ref):
    j = pl.program_id(0)
    t = _ssp(jnp.dot(h_ref[...], l1w_ref[...],
                     preferred_element_type=jnp.float32) + l1b_ref[...])
    y = jnp.dot(t, l2w_ref[...], preferred_element_type=jnp.float32) + l2b_ref[...]
    g = jax.lax.broadcasted_iota(jnp.int32, (_NG, 1), 0)
    onehot_t = (batchT_ref[...] == g).astype(jnp.float32)

    @pl.when(j == 0)
    def _():
        out_ref[...] = jnp.zeros_like(out_ref)

    out_ref[...] += jnp.dot(onehot_t, y, preferred_element_type=jnp.float32)


def _full(shape):
    return pl.BlockSpec(shape, lambda b: tuple(0 for _ in shape))


_PAR = pltpu.CompilerParams(dimension_semantics=("parallel",))


def kernel(z, pos, batch, emb, mlp_w1, mlp_b1, mlp_w2, mlp_b2, cf_w1, cf_w2,
           cf_b2, lin_w, lin_b, lin1_w, lin1_b, lin2_w, lin2_b):
    batch = batch.astype(jnp.int32)
    z2 = z.astype(jnp.int32).reshape(_N, 1)
    batch_t = batch.reshape(1, _N)

    gids = jnp.arange(_NG, dtype=jnp.int32)
    seg_lo = jnp.searchsorted(batch, gids, side='left').astype(jnp.int32)
    seg_hi = jnp.searchsorted(batch, gids, side='right').astype(jnp.int32)
    node_lo = seg_lo[batch]
    node_len = seg_hi[batch] - node_lo
    node_nw = (node_len + _WP - 1) // _WP

    pos4 = jnp.concatenate(
        [pos, batch.astype(jnp.float32).reshape(_N, 1) * 1000.0], axis=1)

    nb = _N // _BLK
    nd = _N // _DB

    h, xl = pl.pallas_call(
        _embed_kernel,
        grid=(nd,),
        in_specs=[
            pl.BlockSpec((_DB, 1), lambda j: (j, 0)),
            _full((100, _HIDDEN)),
            _full((_HIDDEN, _FILTERS)),
        ],
        out_specs=[
            pl.BlockSpec((_DB, _HIDDEN), lambda j: (j, 0)),
            pl.BlockSpec((_DB, _FILTERS), lambda j: (j, 0)),
        ],
        out_shape=[
            jax.ShapeDtypeStruct((_N, _HIDDEN), jnp.float32),
            jax.ShapeDtypeStruct((_N, _FILTERS), jnp.float32),
        ],
        compiler_params=_PAR,
    )(z2, emb, cf_w1[0])

    msg_call = pl.pallas_call(
        _msg_kernel,
        grid=(nb,),
        in_specs=[
            pl.BlockSpec(memory_space=pltpu.SMEM),
            pl.BlockSpec(memory_space=pltpu.SMEM),
            _full((_N, 4)),
            _full((_N, _FILTERS)),
            _full((_NUM_G, _FILTERS)),
            _full((1, _FILTERS)),
            _full((_FILTERS, _FILTERS)),
            _full((1, _FILTERS)),
        ],
        out_specs=pl.BlockSpec((_BLK, _FILTERS), lambda b: (b, 0)),
        out_shape=jax.ShapeDtypeStruct((_N, _FILTERS), jnp.float32),
        compiler_params=_PAR,
    )

    post_call = pl.pallas_call(
        _post_kernel,
        grid=(nd,),
        in_specs=[
            pl.BlockSpec((_DB, _FILTERS), lambda j: (j, 0)),
            pl.BlockSpec((_DB, _HIDDEN), lambda j: (j, 0)),
            _full((_FILTERS, _HIDDEN)),
            _full((1, _HIDDEN)),
            _full((_HIDDEN, _HIDDEN)),
            _full((1, _HIDDEN)),
            _full((_HIDDEN, _FILTERS)),
        ],
        out_specs=[
            pl.BlockSpec((_DB, _HIDDEN), lambda j: (j, 0)),
            pl.BlockSpec((_DB, _FILTERS), lambda j: (j, 0)),
        ],
        out_shape=[
            jax.ShapeDtypeStruct((_N, _HIDDEN), jnp.float32),
            jax.ShapeDtypeStruct((_N, _FILTERS), jnp.float32),
        ],
        compiler_params=_PAR,
    )

    for i in range(_NUM_INT):
        agg = msg_call(node_lo, node_nw, pos4, xl,
                       mlp_w1[i], mlp_b1[i].reshape(1, _FILTERS),
                       mlp_w2[i], mlp_b2[i].reshape(1, _FILTERS))
        w1n = cf_w1[(i + 1) % _NUM_INT]
        h, xl = post_call(agg, h, cf_w2[i], cf_b2[i].reshape(1, _HIDDEN),
                          lin_w[i], lin_b[i].reshape(1, _HIDDEN), w1n)

    out = pl.pallas_call(
        _head_kernel,
        grid=(nd,),
        in_specs=[
            pl.BlockSpec((_DB, _HIDDEN), lambda j: (j, 0)),
            _full((_HIDDEN, _HIDDEN // 2)),
            _full((1, _HIDDEN // 2)),
            _full((_HIDDEN // 2, 1)),
            _full((1, 1)),
            pl.BlockSpec((1, _DB), lambda j: (0, j)),
        ],
        out_specs=pl.BlockSpec((_NG, 1), lambda j: (0, 0)),
        out_shape=jax.ShapeDtypeStruct((_NG, 1), jnp.float32),
    )(h, lin1_w, lin1_b.reshape(1, _HIDDEN // 2),
      lin2_w, lin2_b.reshape(1, 1), batch_t)

    return out
